# Initial kernel scaffold; baseline (speedup 1.0000x reference)
import jax
import jax.numpy as jnp
from jax.experimental import pallas as pl


def kernel(x, edge_index, batch, params):
    raise NotImplementedError("write your pallas kernel here")


# speedup vs baseline: 7.4188x; 7.4188x over previous
import functools
import math

import jax
import jax.numpy as jnp
from jax import lax
from jax.experimental import pallas as pl
from jax.experimental.pallas import tpu as pltpu
from jax.experimental.pallas import tpu_sc as plsc

N = 10000
E = 160000
G = 64
N_PAD = 10240
NC, NS, LANES = 2, 16, 16
NW = NC * NS

E_GAT = 172032
E_TRN = 163840

_DUMMY = N


def _mesh():
    return plsc.VectorSubcoreMesh(core_axis_name="c", subcore_axis_name="s",
                                  num_cores=NC, num_subcores=NS)


def _leaky(x):
    return jnp.where(x >= 0, x, 0.2 * x)


def _mm_body(a_ref, w_ref, b_ref, o_ref):
    o_ref[...] = (jnp.dot(a_ref[...], w_ref[...],
                          preferred_element_type=jnp.float32) + b_ref[...])


def mm_flat(a, w, b, bm=512, bn=256):
    M, K = a.shape
    Nout = w.shape[1]
    bn = min(bn, Nout)
    return pl.pallas_call(
        _mm_body,
        grid=(M // bm, Nout // bn),
        in_specs=[pl.BlockSpec((bm, K), lambda i, j: (i, 0)),
                  pl.BlockSpec((K, bn), lambda i, j: (0, j)),
                  pl.BlockSpec((1, bn), lambda i, j: (0, j))],
        out_specs=pl.BlockSpec((bm, bn), lambda i, j: (i, j)),
        out_shape=jax.ShapeDtypeStruct((M, Nout), jnp.float32),
    )(a, w, b.reshape(1, Nout))


def _mmc_body(a_ref, w_ref, b_ref, o_ref):
    o_ref[0] = (jnp.dot(a_ref[...], w_ref[...],
                        preferred_element_type=jnp.float32) + b_ref[...])


def mm_chunk(a, w, b, bm=512):
    M, K = a.shape
    Nout = w.shape[1]
    kc = Nout // 128
    return pl.pallas_call(
        _mmc_body,
        grid=(kc, M // bm),
        in_specs=[pl.BlockSpec((bm, K), lambda j, i: (i, 0)),
                  pl.BlockSpec((K, 128), lambda j, i: (0, j)),
                  pl.BlockSpec((1, 128), lambda j, i: (0, j))],
        out_specs=pl.BlockSpec((1, bm, 128), lambda j, i: (j, i, 0)),
        out_shape=jax.ShapeDtypeStruct((kc, M, 128), jnp.float32),
    )(a, w, b.reshape(1, Nout))


def _norm_rows(num_ref, den_ref, H, C, Kc, bm):
    d = jnp.sum(den_ref[...], axis=0)[:, :H] + 1e-16
    cols = []
    for k in range(Kc):
        blk = num_ref[k]
        if C >= 128:
            hd = (k * 128) // C
            cols.append(blk / d[:, hd:hd + 1])
        else:
            nh = 128 // C
            h0 = (k * 128) // C
            dv = d[:, h0:h0 + nh]
            div = jnp.repeat(dv, C, axis=1)
            cols.append(blk / div)
    return jnp.concatenate(cols, axis=1)


def gat_post(num, den, b, bn_g, bn_b, bn_m, bn_v, H, C, concat, bm=256):
    Kc, M, _ = num.shape
    Dout = H * C if concat else C
    pp = jnp.stack([b, bn_g, bn_b, bn_m, bn_v])

    def body(num_ref, den_ref, pp_ref, o_ref):
        out = _norm_rows(num_ref, den_ref, H, C, Kc, bm)
        if not concat:
            out = out.reshape(bm, H, C).mean(axis=1)
        out = out + pp_ref[0]
        out = (out - pp_ref[3]) / jnp.sqrt(pp_ref[4] + 1e-5) * pp_ref[1] \
            + pp_ref[2]
        o_ref[...] = _leaky(out)

    return pl.pallas_call(
        body,
        grid=(M // bm,),
        in_specs=[pl.BlockSpec((Kc, bm, 128), lambda i: (0, i, 0)),
                  pl.BlockSpec((NC, bm, 128), lambda i: (0, i, 0)),
                  pl.BlockSpec((5, Dout), lambda i: (0, 0))],
        out_specs=pl.BlockSpec((bm, Dout), lambda i: (i, 0)),
        out_shape=jax.ShapeDtypeStruct((M, Dout), jnp.float32),
    )(num, den, pp)


def trans_post(num, den, skip, identity, H, C, concat, bm=256):
    Kc, M, _ = num.shape
    Dout = H * C if concat else C
    have_id = identity is not None

    def body(*refs):
        if have_id:
            num_ref, den_ref, skip_ref, id_ref, o_ref = refs
        else:
            num_ref, den_ref, skip_ref, o_ref = refs
        out = _norm_rows(num_ref, den_ref, H, C, Kc, bm)
        if not concat:
            out = out.reshape(bm, H, C).mean(axis=1)
        out = _leaky(out + skip_ref[...])
        if have_id:
            out = out + id_ref[...]
        o_ref[...] = out

    in_specs = [pl.BlockSpec((Kc, bm, 128), lambda i: (0, i, 0)),
                pl.BlockSpec((NC, bm, 128), lambda i: (0, i, 0)),
                pl.BlockSpec((bm, Dout), lambda i: (i, 0))]
    args = [num, den, skip]
    if have_id:
        in_specs.append(pl.BlockSpec((bm, Dout), lambda i: (i, 0)))
        args.append(identity)
    return pl.pallas_call(
        body,
        grid=(M // bm,),
        in_specs=in_specs,
        out_specs=pl.BlockSpec((bm, Dout), lambda i: (i, 0)),
        out_shape=jax.ShapeDtypeStruct((M, Dout), jnp.float32),
    )(*args)


def pool_tc(h, batch2d, bmin, bmax, bm=256):
    M, D = h.shape
    nblk = M // bm

    def body(bmin_ref, bmax_ref, h_ref, b_ref, mx_ref, sm_ref, ct_ref):
        i = pl.program_id(0)
        g = pl.program_id(1)

        @pl.when((i == 0) & (g == 0))
        def _init():
            mx_ref[...] = jnp.full((G, D), -1e30, jnp.float32)
            sm_ref[...] = jnp.zeros((G, D), jnp.float32)
            ct_ref[...] = jnp.zeros((G, 128), jnp.float32)

        @pl.when((g >= bmin_ref[i]) & (g <= bmax_ref[i]))
        def _acc():
            mask = b_ref[...] == g
            hb = h_ref[...]
            mxv = jnp.max(jnp.where(mask, hb, -1e30), axis=0, keepdims=True)
            smv = jnp.sum(jnp.where(mask, hb, 0.0), axis=0, keepdims=True)
            ctv = jnp.sum(mask.astype(jnp.float32))
            mx_ref[pl.ds(g, 1), :] = jnp.maximum(mx_ref[pl.ds(g, 1), :], mxv)
            sm_ref[pl.ds(g, 1), :] = sm_ref[pl.ds(g, 1), :] + smv
            ct_ref[pl.ds(g, 1), :] = ct_ref[pl.ds(g, 1), :] + ctv

    grid_spec = pltpu.PrefetchScalarGridSpec(
        num_scalar_prefetch=2,
        grid=(nblk, G),
        in_specs=[pl.BlockSpec((bm, D), lambda i, g, *_: (i, 0)),
                  pl.BlockSpec((bm, 1), lambda i, g, *_: (i, 0))],
        out_specs=[pl.BlockSpec((G, D), lambda i, g, *_: (0, 0)),
                   pl.BlockSpec((G, D), lambda i, g, *_: (0, 0)),
                   pl.BlockSpec((G, 128), lambda i, g, *_: (0, 0))])
    return pl.pallas_call(
        body, grid_spec=grid_spec,
        out_shape=[jax.ShapeDtypeStruct((G, D), jnp.float32),
                   jax.ShapeDtypeStruct((G, D), jnp.float32),
                   jax.ShapeDtypeStruct((G, 128), jnp.float32)],
    )(bmin, bmax, h, batch2d)


def final_mlp(mx, sm, cnt, ln_g, ln_b, w1, b1, w2, b2):
    D = mx.shape[-1]
    H2 = w2.shape[1]

    def body(mx_ref, sm_ref, cnt_ref, lng_ref, lnb_ref,
             w1_ref, b1_ref, w2_ref, b2_ref, o_ref):
        c = cnt_ref[:, 0:1]
        meanp = sm_ref[...] / jnp.maximum(c, 1.0)
        maxp = jnp.where(c > 0, mx_ref[...], 0.0)
        z = jnp.concatenate([maxp, meanp], axis=1)
        mu = jnp.mean(z, axis=-1, keepdims=True)
        var = jnp.mean((z - mu) ** 2, axis=-1, keepdims=True)
        z = (z - mu) / jnp.sqrt(var + 1e-5) * lng_ref[...] + lnb_ref[...]
        z = _leaky(jnp.dot(z, w1_ref[...],
                           preferred_element_type=jnp.float32) + b1_ref[...])
        z = jnp.dot(z, w2_ref[...],
                    preferred_element_type=jnp.float32) + b2_ref[...]
        o_ref[...] = jax.nn.sigmoid(z)

    return pl.pallas_call(
        body,
        out_shape=jax.ShapeDtypeStruct((G, H2), jnp.float32),
    )(mx, sm, cnt, ln_g.reshape(1, 2 * D), ln_b.reshape(1, 2 * D),
      w1, b1.reshape(1, w1.shape[1]), w2, b2.reshape(1, H2))


def _barrier():
    plsc.subcore_barrier()


def _axis_ids():
    return lax.axis_index("c"), lax.axis_index("s")


def _vperm(v, idx):
    return lax.gather(
        v, idx.reshape(16, 1),
        lax.GatherDimensionNumbers(offset_dims=(), collapsed_slice_dims=(0,),
                                   start_index_map=(0,)),
        (1,), mode=lax.GatherScatterMode.PROMISE_IN_BOUNDS)


def _lane_sum(v):
    iota = lax.broadcasted_iota(jnp.int32, (16,), 0)
    for sh in (8, 4, 2, 1):
        v = v + _vperm(v, iota ^ sh)
    return v


def _gather_rows(tab_h, idx_ref, out_ref, sem):
    pltpu.async_copy(tab_h.at[idx_ref], out_ref, sem).wait()


def _scatter_add_rows(src_ref, base_ref, idx_ref):
    pltpu.sync_copy(src_ref, base_ref.at[idx_ref], add=True)


def _zero_vec_buf(buf, rows):
    def zi(i, _):
        buf[i] = jnp.zeros((16,), jnp.float32)
        return 0
    lax.fori_loop(0, rows, zi, 0, unroll=False)


@functools.lru_cache(maxsize=None)
def _make_gat_w(e_pad, H):
    epw = e_pad // NW
    B = 64
    nblk = epw // B
    zsl = N_PAD // NS

    @functools.partial(
        pl.kernel, mesh=_mesh(),
        out_type=[jax.ShapeDtypeStruct((e_pad, 16), jnp.float32),
                  jax.ShapeDtypeStruct((NC, N_PAD, 128), jnp.float32)],
        scratch_types=[
            pltpu.VMEM((64,), jnp.int32),
            pltpu.VMEM((64,), jnp.int32),
            pltpu.VMEM((64, 128), jnp.float32),
            pltpu.VMEM((64, 128), jnp.float32),
            pltpu.VMEM((64, 16), jnp.float32),
            pltpu.VMEM((64, 128), jnp.float32),
            pltpu.VMEM((16, 128), jnp.float32),
            pltpu.VMEM_SHARED((N_PAD, 128), jnp.float32),
            pltpu.SemaphoreType.DMA,
            pltpu.SemaphoreType.DMA,
        ])
    def k(src_h, dst_h, sc_h, w_h, den_h,
          srcv, dstv, ur, vr, wb, wwide, zb, dacc, sem1, sem2):
        cid, sid = _axis_ids()
        wid = sid * NC + cid

        def zrow(i, _):
            for j in range(8):
                zb[i, pl.ds(j * 16, 16)] = jnp.zeros((16,), jnp.float32)
            return 0
        lax.fori_loop(0, 16, zrow, 0, unroll=False)

        def zwide(i, _):
            for j in range(8):
                wwide[i, pl.ds(j * 16, 16)] = jnp.zeros((16,), jnp.float32)
            return 0
        lax.fori_loop(0, 64, zwide, 0, unroll=False)

        def zcp(zi, _):
            pltpu.sync_copy(zb, dacc.at[pl.ds(sid * zsl + zi * 16, 16)])
            return 0
        lax.fori_loop(0, zsl // 16, zcp, 0, unroll=False)
        _barrier()
        base0 = wid * epw

        def blk(bi, _):
            base = base0 + bi * B
            pltpu.sync_copy(src_h.at[pl.ds(base, B)], srcv)
            pltpu.sync_copy(dst_h.at[pl.ds(base, B)], dstv)
            _gather_rows(sc_h, srcv, ur, sem1)
            _gather_rows(sc_h, dstv, vr, sem2)

            def per_edge(e, _):
                a = ur[e, pl.ds(0, 16)] + vr[e, pl.ds(16, 16)]
                w = jnp.exp(jnp.where(a >= 0, a, 0.2 * a))
                wb[e] = w
                wwide[e, pl.ds(0, 16)] = w
                return 0
            lax.fori_loop(0, B, per_edge, 0, unroll=False)
            pltpu.sync_copy(wb, w_h.at[pl.ds(base, B)])
            _scatter_add_rows(wwide, dacc, dstv)
            return 0
        lax.fori_loop(0, nblk, blk, 0, unroll=False)
        _barrier()
        pltpu.sync_copy(dacc.at[pl.ds(sid * zsl, zsl)],
                        den_h.at[cid, pl.ds(sid * zsl, zsl)])

    return k


@functools.lru_cache(maxsize=None)
def _make_trans_w(e_pad, H, C, B):
    D = H * C
    epw = e_pad // NW
    nblk = epw // B
    zsl = N_PAD // NS
    scale = 1.0 / math.sqrt(float(C))

    @functools.partial(
        pl.kernel, mesh=_mesh(),
        out_type=[jax.ShapeDtypeStruct((e_pad, 16), jnp.float32),
                  jax.ShapeDtypeStruct((NC, N_PAD, 128), jnp.float32)],
        scratch_types=[
            pltpu.VMEM((B,), jnp.int32),
            pltpu.VMEM((B,), jnp.int32),
            pltpu.VMEM((B, D), jnp.float32),
            pltpu.VMEM((B, D), jnp.float32),
            pltpu.VMEM((B, 16), jnp.float32),
            pltpu.VMEM((B, 128), jnp.float32),
            pltpu.VMEM((16, 128), jnp.float32),
            pltpu.VMEM_SHARED((N_PAD, 128), jnp.float32),
            pltpu.SemaphoreType.DMA,
            pltpu.SemaphoreType.DMA,
        ])
    def k(src_h, dst_h, q_h, k_h, w_h, den_h,
          srcv, dstv, qr, kr, wb, wwide, zb, dacc, sem1, sem2):
        cid, sid = _axis_ids()
        wid = sid * NC + cid
        iota = lax.broadcasted_iota(jnp.int32, (16,), 0)

        def zrow(i, _):
            for j in range(8):
                zb[i, pl.ds(j * 16, 16)] = jnp.zeros((16,), jnp.float32)
            return 0
        lax.fori_loop(0, 16, zrow, 0, unroll=False)

        def zwide(i, _):
            for j in range(8):
                wwide[i, pl.ds(j * 16, 16)] = jnp.zeros((16,), jnp.float32)
            return 0
        lax.fori_loop(0, B, zwide, 0, unroll=False)

        def zcp(zi, _):
            pltpu.sync_copy(zb, dacc.at[pl.ds(sid * zsl + zi * 16, 16)])
            return 0
        lax.fori_loop(0, zsl // 16, zcp, 0, unroll=False)
        _barrier()
        base0 = wid * epw

        def blk(bi, _):
            base = base0 + bi * B
            pltpu.sync_copy(src_h.at[pl.ds(base, B)], srcv)
            pltpu.sync_copy(dst_h.at[pl.ds(base, B)], dstv)
            _gather_rows(q_h, dstv, qr, sem1)
            _gather_rows(k_h, srcv, kr, sem2)

            def per_edge(e, _):
                w = jnp.zeros((16,), jnp.float32)
                for hd in range(H):
                    acc = jnp.zeros((16,), jnp.float32)
                    for j in range(C // 16):
                        off = hd * C + j * 16
                        acc = acc + qr[e, pl.ds(off, 16)] * kr[e, pl.ds(off, 16)]
                    t = _lane_sum(acc) * scale
                    w = jnp.where(iota == hd, t, w)
                w = jnp.exp(jnp.where(iota < H, w, jnp.zeros((16,), jnp.float32)))
                w = jnp.where(iota < H, w, jnp.zeros((16,), jnp.float32))
                wb[e] = w
                wwide[e, pl.ds(0, 16)] = w
                return 0
            lax.fori_loop(0, B, per_edge, 0, unroll=False)
            pltpu.sync_copy(wb, w_h.at[pl.ds(base, B)])
            _scatter_add_rows(wwide, dacc, dstv)
            return 0
        lax.fori_loop(0, nblk, blk, 0, unroll=False)
        _barrier()
        pltpu.sync_copy(dacc.at[pl.ds(sid * zsl, zsl)],
                        den_h.at[cid, pl.ds(sid * zsl, zsl)])

    return k


@functools.lru_cache(maxsize=None)
def _make_agg(e_pad, K, C):
    KPC = K // NC
    epw = e_pad // NS
    B = 64
    nblk = epw // B
    zsl = N_PAD // NS
    logc = int(math.log2(C))

    @functools.partial(
        pl.kernel, mesh=_mesh(),
        out_type=jax.ShapeDtypeStruct((K, N_PAD, 128), jnp.float32),
        scratch_types=[
            pltpu.VMEM((B,), jnp.int32),
            pltpu.VMEM((B,), jnp.int32),
            pltpu.VMEM((B,), jnp.int32),
            pltpu.VMEM((B, 128), jnp.float32),
            pltpu.VMEM((B, 16), jnp.float32),
            pltpu.VMEM((16, 128), jnp.float32),
            pltpu.VMEM_SHARED((N_PAD, 128), jnp.float32),
            pltpu.SemaphoreType.DMA,
        ])
    def k(src_h, dst_h, w_h, v_h, out_h,
          srcv, dstv, idxv, rows, wb, zb, acc, sem):
        cid, sid = _axis_ids()

        def zrow(i, _):
            for j in range(8):
                zb[i, pl.ds(j * 16, 16)] = jnp.zeros((16,), jnp.float32)
            return 0
        lax.fori_loop(0, 16, zrow, 0, unroll=False)

        for kk in range(KPC):
            kchunk = cid * KPC + kk
            kbase = kchunk * N_PAD

            def zcp(zi, _):
                pltpu.sync_copy(zb, acc.at[pl.ds(sid * zsl + zi * 16, 16)])
                return 0
            lax.fori_loop(0, zsl // 16, zcp, 0, unroll=False)
            _barrier()

            def blk(bi, _):
                base = sid * epw + bi * B
                pltpu.sync_copy(src_h.at[pl.ds(base, B)], srcv)
                pltpu.sync_copy(dst_h.at[pl.ds(base, B)], dstv)
                pltpu.sync_copy(w_h.at[pl.ds(base, B)], wb)

                def mkidx(i, _):
                    idxv[pl.ds(i * 16, 16)] = srcv[pl.ds(i * 16, 16)] + kbase
                    return 0
                lax.fori_loop(0, B // 16, mkidx, 0, unroll=True)
                _gather_rows(v_h, idxv, rows, sem)

                def per_edge(e, _):
                    wv = wb[e]
                    for j in range(8):
                        hdj = (kchunk * 128 + j * 16) >> logc
                        m = _vperm(wv, jnp.full((16,), hdj, jnp.int32))
                        rows[e, pl.ds(j * 16, 16)] = rows[e, pl.ds(j * 16, 16)] * m
                    return 0
                lax.fori_loop(0, B, per_edge, 0, unroll=False)
                _scatter_add_rows(rows, acc, dstv)
                return 0
            lax.fori_loop(0, nblk, blk, 0, unroll=False)
            _barrier()
            pltpu.sync_copy(acc.at[pl.ds(sid * zsl, zsl)],
                            out_h.at[kchunk, pl.ds(sid * zsl, zsl)])
            _barrier()

    return k


def _score_weights(W, a_s, a_d, H, C):
    Din = W.shape[0]
    ws = (W.reshape(Din, H, C) * a_s[None]).sum(-1)
    wd = (W.reshape(Din, H, C) * a_d[None]).sum(-1)
    Wsc = jnp.zeros((Din, 128), jnp.float32)
    Wsc = Wsc.at[:, 0:H].set(ws).at[:, 16:16 + H].set(wd)
    return Wsc


def _gat_layer(h, src, dst, W, a_s, a_d, b, bn_g, bn_b, bn_m, bn_v,
               H, C, concat):
    Kc = (H * C) // 128
    hp = mm_chunk(h, W, jnp.zeros((H * C,), jnp.float32))
    sc = mm_flat(h, _score_weights(W, a_s, a_d, H, C),
                 jnp.zeros((128,), jnp.float32), bn=128)
    w, den = _make_gat_w(E_GAT, H)(src, dst, sc)
    num = _make_agg(E_GAT, Kc, C)(src, dst, w, hp.reshape(Kc * N_PAD, 128))
    return gat_post(num, den, b, bn_g, bn_b, bn_m, bn_v, H, C, concat)


def _trans_layer(h, src, dst, Wq, bq, Wk, bk, Wv, bv, Ws, bs,
                 H, C, concat, identity):
    Kc = (H * C) // 128
    q = mm_flat(h, Wq, bq)
    kt = mm_flat(h, Wk, bk)
    v = mm_chunk(h, Wv, bv)
    skip = mm_flat(h, Ws, bs)
    B = 16
    w, den = _make_trans_w(E_TRN, H, C, B)(src, dst, q, kt)
    num = _make_agg(E_TRN, Kc, C)(src, dst, w, v.reshape(Kc * N_PAD, 128))
    return trans_post(num, den, skip, identity, H, C, concat)


def kernel(x, edge_index, batch, params):
    p = params
    src = edge_index[0].astype(jnp.int32)
    dst = edge_index[1].astype(jnp.int32)
    loop = jnp.arange(N, dtype=jnp.int32)
    fill_g = jnp.full((E_GAT - E - N,), _DUMMY, jnp.int32)
    src_g = jnp.concatenate([src, loop, fill_g])
    dst_g = jnp.concatenate([dst, loop, fill_g])
    fill_t = jnp.full((E_TRN - E,), _DUMMY, jnp.int32)
    src_t = jnp.concatenate([src, fill_t])
    dst_t = jnp.concatenate([dst, fill_t])

    xp = jnp.zeros((N_PAD, x.shape[1]), jnp.float32).at[:N].set(x)
    batch_p = jnp.concatenate(
        [batch.astype(jnp.int32), jnp.full((N_PAD - N,), G, jnp.int32)])

    h1 = _gat_layer(xp, src_g, dst_g, p['W1'], p['as1'], p['ad1'], p['b1'],
                    p['bn1_g'], p['bn1_b'], p['bn1_m'], p['bn1_v'],
                    8, 32, True)
    h2 = _gat_layer(h1, src_g, dst_g, p['W2'], p['as2'], p['ad2'], p['b2'],
                    p['bn2_g'], p['bn2_b'], p['bn2_m'], p['bn2_v'],
                    8, 64, True)
    h3 = _gat_layer(h2, src_g, dst_g, p['W3'], p['as3'], p['ad3'], p['b3'],
                    p['bn3_g'], p['bn3_b'], p['bn3_m'], p['bn3_v'],
                    4, 256, False)
    h4 = _trans_layer(h3, src_t, dst_t,
                      p['t1_Wq'], p['t1_bq'], p['t1_Wk'], p['t1_bk'],
                      p['t1_Wv'], p['t1_bv'], p['t1_Ws'], p['t1_bs'],
                      8, 64, True, None)
    h5 = _trans_layer(h4, src_t, dst_t,
                      p['t2_Wq'], p['t2_bq'], p['t2_Wk'], p['t2_bk'],
                      p['t2_Wv'], p['t2_bv'], p['t2_Ws'], p['t2_bs'],
                      4, 256, False, h3)

    bmin = batch_p[0::256]
    bmax = batch_p[255::256]
    mx, sm, cnt = pool_tc(h5, batch_p.reshape(N_PAD, 1), bmin, bmax)
    return final_mlp(mx, sm, cnt,
                     p['ln_g'], p['ln_b'], p['fc1_W'], p['fc1_b'],
                     p['fc2_W'], p['fc2_b'])


# speedup vs baseline: 7.8577x; 1.0592x over previous
import functools
import math

import jax
import jax.numpy as jnp
from jax import lax
from jax.experimental import pallas as pl
from jax.experimental.pallas import tpu as pltpu
from jax.experimental.pallas import tpu_sc as plsc

N = 10000
E = 160000
G = 64
N_PAD = 10240
NC, NS, LANES = 2, 16, 16
NW = NC * NS

E_GAT = 172032
E_TRN = 163840

_DUMMY = N


def _mesh():
    return plsc.VectorSubcoreMesh(core_axis_name="c", subcore_axis_name="s",
                                  num_cores=NC, num_subcores=NS)


def _leaky(x):
    return jnp.where(x >= 0, x, 0.2 * x)


def _mm_body(a_ref, w_ref, b_ref, o_ref):
    o_ref[...] = (jnp.dot(a_ref[...], w_ref[...],
                          preferred_element_type=jnp.float32) + b_ref[...])


def mm_flat(a, w, b, bm=512, bn=256):
    M, K = a.shape
    Nout = w.shape[1]
    bn = min(bn, Nout)
    return pl.pallas_call(
        _mm_body,
        grid=(M // bm, Nout // bn),
        in_specs=[pl.BlockSpec((bm, K), lambda i, j: (i, 0)),
                  pl.BlockSpec((K, bn), lambda i, j: (0, j)),
                  pl.BlockSpec((1, bn), lambda i, j: (0, j))],
        out_specs=pl.BlockSpec((bm, bn), lambda i, j: (i, j)),
        out_shape=jax.ShapeDtypeStruct((M, Nout), jnp.float32),
    )(a, w, b.reshape(1, Nout))


def _mmc_body(a_ref, w_ref, b_ref, o_ref):
    o_ref[0] = (jnp.dot(a_ref[...], w_ref[...],
                        preferred_element_type=jnp.float32) + b_ref[...])


def mm_chunk(a, w, b, bm=512):
    M, K = a.shape
    Nout = w.shape[1]
    kc = Nout // 128
    return pl.pallas_call(
        _mmc_body,
        grid=(kc, M // bm),
        in_specs=[pl.BlockSpec((bm, K), lambda j, i: (i, 0)),
                  pl.BlockSpec((K, 128), lambda j, i: (0, j)),
                  pl.BlockSpec((1, 128), lambda j, i: (0, j))],
        out_specs=pl.BlockSpec((1, bm, 128), lambda j, i: (j, i, 0)),
        out_shape=jax.ShapeDtypeStruct((kc, M, 128), jnp.float32),
    )(a, w, b.reshape(1, Nout))


def _norm_rows(num_ref, den_ref, H, C, Kc, bm):
    d = jnp.sum(den_ref[...], axis=0)[:, :H] + 1e-16
    cols = []
    for k in range(Kc):
        blk = num_ref[k]
        if C >= 128:
            hd = (k * 128) // C
            cols.append(blk / d[:, hd:hd + 1])
        else:
            nh = 128 // C
            h0 = (k * 128) // C
            dv = d[:, h0:h0 + nh]
            div = jnp.repeat(dv, C, axis=1)
            cols.append(blk / div)
    return jnp.concatenate(cols, axis=1)


def gat_post(num, den, b, bn_g, bn_b, bn_m, bn_v, H, C, concat, bm=256):
    Kc, M, _ = num.shape
    Dout = H * C if concat else C
    pp = jnp.stack([b, bn_g, bn_b, bn_m, bn_v])

    def body(num_ref, den_ref, pp_ref, o_ref):
        out = _norm_rows(num_ref, den_ref, H, C, Kc, bm)
        if not concat:
            out = out.reshape(bm, H, C).mean(axis=1)
        out = out + pp_ref[0]
        out = (out - pp_ref[3]) / jnp.sqrt(pp_ref[4] + 1e-5) * pp_ref[1] \
            + pp_ref[2]
        o_ref[...] = _leaky(out)

    return pl.pallas_call(
        body,
        grid=(M // bm,),
        in_specs=[pl.BlockSpec((Kc, bm, 128), lambda i: (0, i, 0)),
                  pl.BlockSpec((NC, bm, 128), lambda i: (0, i, 0)),
                  pl.BlockSpec((5, Dout), lambda i: (0, 0))],
        out_specs=pl.BlockSpec((bm, Dout), lambda i: (i, 0)),
        out_shape=jax.ShapeDtypeStruct((M, Dout), jnp.float32),
    )(num, den, pp)


def trans_post(num, den, skip, identity, H, C, concat, bm=256):
    Kc, M, _ = num.shape
    Dout = H * C if concat else C
    have_id = identity is not None

    def body(*refs):
        if have_id:
            num_ref, den_ref, skip_ref, id_ref, o_ref = refs
        else:
            num_ref, den_ref, skip_ref, o_ref = refs
        out = _norm_rows(num_ref, den_ref, H, C, Kc, bm)
        if not concat:
            out = out.reshape(bm, H, C).mean(axis=1)
        out = _leaky(out + skip_ref[...])
        if have_id:
            out = out + id_ref[...]
        o_ref[...] = out

    in_specs = [pl.BlockSpec((Kc, bm, 128), lambda i: (0, i, 0)),
                pl.BlockSpec((NC, bm, 128), lambda i: (0, i, 0)),
                pl.BlockSpec((bm, Dout), lambda i: (i, 0))]
    args = [num, den, skip]
    if have_id:
        in_specs.append(pl.BlockSpec((bm, Dout), lambda i: (i, 0)))
        args.append(identity)
    return pl.pallas_call(
        body,
        grid=(M // bm,),
        in_specs=in_specs,
        out_specs=pl.BlockSpec((bm, Dout), lambda i: (i, 0)),
        out_shape=jax.ShapeDtypeStruct((M, Dout), jnp.float32),
    )(*args)


def pool_tc(h, batch2d, bmin, bmax, bm=256):
    M, D = h.shape
    nblk = M // bm

    def body(bmin_ref, bmax_ref, h_ref, b_ref, mx_ref, sm_ref, ct_ref):
        i = pl.program_id(0)
        g = pl.program_id(1)

        @pl.when((i == 0) & (g == 0))
        def _init():
            mx_ref[...] = jnp.full((G, D), -1e30, jnp.float32)
            sm_ref[...] = jnp.zeros((G, D), jnp.float32)
            ct_ref[...] = jnp.zeros((G, 128), jnp.float32)

        @pl.when((g >= bmin_ref[i]) & (g <= bmax_ref[i]))
        def _acc():
            mask = b_ref[...] == g
            hb = h_ref[...]
            mxv = jnp.max(jnp.where(mask, hb, -1e30), axis=0, keepdims=True)
            smv = jnp.sum(jnp.where(mask, hb, 0.0), axis=0, keepdims=True)
            ctv = jnp.sum(mask.astype(jnp.float32))
            mx_ref[pl.ds(g, 1), :] = jnp.maximum(mx_ref[pl.ds(g, 1), :], mxv)
            sm_ref[pl.ds(g, 1), :] = sm_ref[pl.ds(g, 1), :] + smv
            ct_ref[pl.ds(g, 1), :] = ct_ref[pl.ds(g, 1), :] + ctv

    grid_spec = pltpu.PrefetchScalarGridSpec(
        num_scalar_prefetch=2,
        grid=(nblk, G),
        in_specs=[pl.BlockSpec((bm, D), lambda i, g, *_: (i, 0)),
                  pl.BlockSpec((bm, 1), lambda i, g, *_: (i, 0))],
        out_specs=[pl.BlockSpec((G, D), lambda i, g, *_: (0, 0)),
                   pl.BlockSpec((G, D), lambda i, g, *_: (0, 0)),
                   pl.BlockSpec((G, 128), lambda i, g, *_: (0, 0))])
    return pl.pallas_call(
        body, grid_spec=grid_spec,
        out_shape=[jax.ShapeDtypeStruct((G, D), jnp.float32),
                   jax.ShapeDtypeStruct((G, D), jnp.float32),
                   jax.ShapeDtypeStruct((G, 128), jnp.float32)],
    )(bmin, bmax, h, batch2d)


def final_mlp(mx, sm, cnt, ln_g, ln_b, w1, b1, w2, b2):
    D = mx.shape[-1]
    H2 = w2.shape[1]

    def body(mx_ref, sm_ref, cnt_ref, lng_ref, lnb_ref,
             w1_ref, b1_ref, w2_ref, b2_ref, o_ref):
        c = cnt_ref[:, 0:1]
        meanp = sm_ref[...] / jnp.maximum(c, 1.0)
        maxp = jnp.where(c > 0, mx_ref[...], 0.0)
        z = jnp.concatenate([maxp, meanp], axis=1)
        mu = jnp.mean(z, axis=-1, keepdims=True)
        var = jnp.mean((z - mu) ** 2, axis=-1, keepdims=True)
        z = (z - mu) / jnp.sqrt(var + 1e-5) * lng_ref[...] + lnb_ref[...]
        z = _leaky(jnp.dot(z, w1_ref[...],
                           preferred_element_type=jnp.float32) + b1_ref[...])
        z = jnp.dot(z, w2_ref[...],
                    preferred_element_type=jnp.float32) + b2_ref[...]
        o_ref[...] = jax.nn.sigmoid(z)

    return pl.pallas_call(
        body,
        out_shape=jax.ShapeDtypeStruct((G, H2), jnp.float32),
    )(mx, sm, cnt, ln_g.reshape(1, 2 * D), ln_b.reshape(1, 2 * D),
      w1, b1.reshape(1, w1.shape[1]), w2, b2.reshape(1, H2))


def _barrier():
    plsc.subcore_barrier()


def _axis_ids():
    return lax.axis_index("c"), lax.axis_index("s")


def _vperm(v, idx):
    return lax.gather(
        v, idx.reshape(16, 1),
        lax.GatherDimensionNumbers(offset_dims=(), collapsed_slice_dims=(0,),
                                   start_index_map=(0,)),
        (1,), mode=lax.GatherScatterMode.PROMISE_IN_BOUNDS)


def _lane_sum(v):
    iota = lax.broadcasted_iota(jnp.int32, (16,), 0)
    for sh in (8, 4, 2, 1):
        v = v + _vperm(v, iota ^ sh)
    return v


def _gather_rows(tab_h, idx_ref, out_ref, sem):
    pltpu.async_copy(tab_h.at[idx_ref], out_ref, sem).wait()


def _gather_rows2(tab1, idx1, out1, sem1, tab2, idx2, out2, sem2):
    d1 = pltpu.async_copy(tab1.at[idx1], out1, sem1)
    d2 = pltpu.async_copy(tab2.at[idx2], out2, sem2)
    d1.wait()
    d2.wait()


def _scatter_add_rows(src_ref, base_ref, idx_ref):
    pltpu.sync_copy(src_ref, base_ref.at[idx_ref], add=True)


def _zero_vec_buf(buf, rows):
    def zi(i, _):
        buf[i] = jnp.zeros((16,), jnp.float32)
        return 0
    lax.fori_loop(0, rows, zi, 0, unroll=False)


@functools.lru_cache(maxsize=None)
def _make_gat_w(e_pad, H):
    epw = e_pad // NW
    B = 64
    nblk = epw // B
    zsl = N_PAD // NS

    @functools.partial(
        pl.kernel, mesh=_mesh(),
        out_type=[jax.ShapeDtypeStruct((e_pad, 16), jnp.float32),
                  jax.ShapeDtypeStruct((NC, N_PAD, 128), jnp.float32)],
        scratch_types=[
            pltpu.VMEM((64,), jnp.int32),
            pltpu.VMEM((64,), jnp.int32),
            pltpu.VMEM((64, 128), jnp.float32),
            pltpu.VMEM((64, 128), jnp.float32),
            pltpu.VMEM((64, 16), jnp.float32),
            pltpu.VMEM((64, 128), jnp.float32),
            pltpu.VMEM((16, 128), jnp.float32),
            pltpu.VMEM_SHARED((N_PAD, 128), jnp.float32),
            pltpu.SemaphoreType.DMA,
            pltpu.SemaphoreType.DMA,
        ])
    def k(src_h, dst_h, sc_h, w_h, den_h,
          srcv, dstv, ur, vr, wb, wwide, zb, dacc, sem1, sem2):
        cid, sid = _axis_ids()
        wid = sid * NC + cid

        def zrow(i, _):
            for j in range(8):
                zb[i, pl.ds(j * 16, 16)] = jnp.zeros((16,), jnp.float32)
            return 0
        lax.fori_loop(0, 16, zrow, 0, unroll=False)

        def zwide(i, _):
            for j in range(8):
                wwide[i, pl.ds(j * 16, 16)] = jnp.zeros((16,), jnp.float32)
            return 0
        lax.fori_loop(0, 64, zwide, 0, unroll=False)

        def zcp(zi, _):
            pltpu.sync_copy(zb, dacc.at[pl.ds(sid * zsl + zi * 16, 16)])
            return 0
        lax.fori_loop(0, zsl // 16, zcp, 0, unroll=False)
        _barrier()
        base0 = wid * epw

        def blk(bi, _):
            base = base0 + bi * B
            pltpu.sync_copy(src_h.at[pl.ds(base, B)], srcv)
            pltpu.sync_copy(dst_h.at[pl.ds(base, B)], dstv)
            _gather_rows2(sc_h, srcv, ur, sem1, sc_h, dstv, vr, sem2)

            def per_edge(e, _):
                a = ur[e, pl.ds(0, 16)] + vr[e, pl.ds(16, 16)]
                w = jnp.exp(jnp.where(a >= 0, a, 0.2 * a))
                wb[e] = w
                wwide[e, pl.ds(0, 16)] = w
                return 0
            lax.fori_loop(0, B, per_edge, 0, unroll=False)
            pltpu.sync_copy(wb, w_h.at[pl.ds(base, B)])
            _scatter_add_rows(wwide, dacc, dstv)
            return 0
        lax.fori_loop(0, nblk, blk, 0, unroll=False)
        _barrier()
        pltpu.sync_copy(dacc.at[pl.ds(sid * zsl, zsl)],
                        den_h.at[cid, pl.ds(sid * zsl, zsl)])

    return k


@functools.lru_cache(maxsize=None)
def _make_trans_w(e_pad, H, C, B):
    D = H * C
    epw = e_pad // NW
    nblk = epw // B
    zsl = N_PAD // NS
    scale = 1.0 / math.sqrt(float(C))

    @functools.partial(
        pl.kernel, mesh=_mesh(),
        out_type=[jax.ShapeDtypeStruct((e_pad, 16), jnp.float32),
                  jax.ShapeDtypeStruct((NC, N_PAD, 128), jnp.float32)],
        scratch_types=[
            pltpu.VMEM((B,), jnp.int32),
            pltpu.VMEM((B,), jnp.int32),
            pltpu.VMEM((B, D), jnp.float32),
            pltpu.VMEM((B, D), jnp.float32),
            pltpu.VMEM((B, 16), jnp.float32),
            pltpu.VMEM((B, 128), jnp.float32),
            pltpu.VMEM((16, 128), jnp.float32),
            pltpu.VMEM_SHARED((N_PAD, 128), jnp.float32),
            pltpu.SemaphoreType.DMA,
            pltpu.SemaphoreType.DMA,
        ])
    def k(src_h, dst_h, q_h, k_h, w_h, den_h,
          srcv, dstv, qr, kr, wb, wwide, zb, dacc, sem1, sem2):
        cid, sid = _axis_ids()
        wid = sid * NC + cid
        iota = lax.broadcasted_iota(jnp.int32, (16,), 0)

        def zrow(i, _):
            for j in range(8):
                zb[i, pl.ds(j * 16, 16)] = jnp.zeros((16,), jnp.float32)
            return 0
        lax.fori_loop(0, 16, zrow, 0, unroll=False)

        def zwide(i, _):
            for j in range(8):
                wwide[i, pl.ds(j * 16, 16)] = jnp.zeros((16,), jnp.float32)
            return 0
        lax.fori_loop(0, B, zwide, 0, unroll=False)

        def zcp(zi, _):
            pltpu.sync_copy(zb, dacc.at[pl.ds(sid * zsl + zi * 16, 16)])
            return 0
        lax.fori_loop(0, zsl // 16, zcp, 0, unroll=False)
        _barrier()
        base0 = wid * epw

        def blk(bi, _):
            base = base0 + bi * B
            pltpu.sync_copy(src_h.at[pl.ds(base, B)], srcv)
            pltpu.sync_copy(dst_h.at[pl.ds(base, B)], dstv)
            _gather_rows2(q_h, dstv, qr, sem1, k_h, srcv, kr, sem2)

            def per_edge(e, _):
                w = jnp.zeros((16,), jnp.float32)
                for hd in range(H):
                    acc = jnp.zeros((16,), jnp.float32)
                    for j in range(C // 16):
                        off = hd * C + j * 16
                        acc = acc + qr[e, pl.ds(off, 16)] * kr[e, pl.ds(off, 16)]
                    t = _lane_sum(acc) * scale
                    w = jnp.where(iota == hd, t, w)
                w = jnp.exp(jnp.where(iota < H, w, jnp.zeros((16,), jnp.float32)))
                w = jnp.where(iota < H, w, jnp.zeros((16,), jnp.float32))
                wb[e] = w
                wwide[e, pl.ds(0, 16)] = w
                return 0
            lax.fori_loop(0, B, per_edge, 0, unroll=False)
            pltpu.sync_copy(wb, w_h.at[pl.ds(base, B)])
            _scatter_add_rows(wwide, dacc, dstv)
            return 0
        lax.fori_loop(0, nblk, blk, 0, unroll=False)
        _barrier()
        pltpu.sync_copy(dacc.at[pl.ds(sid * zsl, zsl)],
                        den_h.at[cid, pl.ds(sid * zsl, zsl)])

    return k


@functools.lru_cache(maxsize=None)
def _make_agg(e_pad, K, C):
    KPC = K // NC
    epw = e_pad // NS
    B = 64
    nblk = epw // B
    zsl = N_PAD // NS
    logc = int(math.log2(C))

    @functools.partial(
        pl.kernel, mesh=_mesh(),
        out_type=jax.ShapeDtypeStruct((K, N_PAD, 128), jnp.float32),
        scratch_types=[
            pltpu.VMEM((B,), jnp.int32),
            pltpu.VMEM((B,), jnp.int32),
            pltpu.VMEM((B,), jnp.int32),
            pltpu.VMEM((B, 128), jnp.float32),
            pltpu.VMEM((B, 16), jnp.float32),
            pltpu.VMEM((16, 128), jnp.float32),
            pltpu.VMEM_SHARED((N_PAD, 128), jnp.float32),
            pltpu.SemaphoreType.DMA,
        ])
    def k(src_h, dst_h, w_h, v_h, out_h,
          srcv, dstv, idxv, rows, wb, zb, acc, sem):
        cid, sid = _axis_ids()

        def zrow(i, _):
            for j in range(8):
                zb[i, pl.ds(j * 16, 16)] = jnp.zeros((16,), jnp.float32)
            return 0
        lax.fori_loop(0, 16, zrow, 0, unroll=False)

        for kk in range(KPC):
            kchunk = cid * KPC + kk
            kbase = kchunk * N_PAD

            def zcp(zi, _):
                pltpu.sync_copy(zb, acc.at[pl.ds(sid * zsl + zi * 16, 16)])
                return 0
            lax.fori_loop(0, zsl // 16, zcp, 0, unroll=False)
            _barrier()

            def blk(bi, _):
                base = sid * epw + bi * B
                pltpu.sync_copy(src_h.at[pl.ds(base, B)], srcv)
                pltpu.sync_copy(dst_h.at[pl.ds(base, B)], dstv)
                pltpu.sync_copy(w_h.at[pl.ds(base, B)], wb)

                def mkidx(i, _):
                    idxv[pl.ds(i * 16, 16)] = srcv[pl.ds(i * 16, 16)] + kbase
                    return 0
                lax.fori_loop(0, B // 16, mkidx, 0, unroll=True)
                _gather_rows(v_h, idxv, rows, sem)

                def per_edge(e, _):
                    wv = wb[e]
                    for j in range(8):
                        hdj = (kchunk * 128 + j * 16) >> logc
                        m = _vperm(wv, jnp.full((16,), hdj, jnp.int32))
                        rows[e, pl.ds(j * 16, 16)] = rows[e, pl.ds(j * 16, 16)] * m
                    return 0
                lax.fori_loop(0, B, per_edge, 0, unroll=False)
                _scatter_add_rows(rows, acc, dstv)
                return 0
            lax.fori_loop(0, nblk, blk, 0, unroll=False)
            _barrier()
            pltpu.sync_copy(acc.at[pl.ds(sid * zsl, zsl)],
                            out_h.at[kchunk, pl.ds(sid * zsl, zsl)])
            _barrier()

    return k


def _score_weights(W, a_s, a_d, H, C):
    Din = W.shape[0]
    ws = (W.reshape(Din, H, C) * a_s[None]).sum(-1)
    wd = (W.reshape(Din, H, C) * a_d[None]).sum(-1)
    Wsc = jnp.zeros((Din, 128), jnp.float32)
    Wsc = Wsc.at[:, 0:H].set(ws).at[:, 16:16 + H].set(wd)
    return Wsc


def _gat_layer(h, src, dst, W, a_s, a_d, b, bn_g, bn_b, bn_m, bn_v,
               H, C, concat):
    Kc = (H * C) // 128
    hp = mm_chunk(h, W, jnp.zeros((H * C,), jnp.float32))
    sc = mm_flat(h, _score_weights(W, a_s, a_d, H, C),
                 jnp.zeros((128,), jnp.float32), bn=128)
    w, den = _make_gat_w(E_GAT, H)(src, dst, sc)
    num = _make_agg(E_GAT, Kc, C)(src, dst, w, hp.reshape(Kc * N_PAD, 128))
    return gat_post(num, den, b, bn_g, bn_b, bn_m, bn_v, H, C, concat)


def _trans_layer(h, src, dst, Wq, bq, Wk, bk, Wv, bv, Ws, bs,
                 H, C, concat, identity):
    Kc = (H * C) // 128
    q = mm_flat(h, Wq, bq)
    kt = mm_flat(h, Wk, bk)
    v = mm_chunk(h, Wv, bv)
    skip = mm_flat(h, Ws, bs)
    B = 16
    w, den = _make_trans_w(E_TRN, H, C, B)(src, dst, q, kt)
    num = _make_agg(E_TRN, Kc, C)(src, dst, w, v.reshape(Kc * N_PAD, 128))
    return trans_post(num, den, skip, identity, H, C, concat)


def kernel(x, edge_index, batch, params):
    p = params
    src = edge_index[0].astype(jnp.int32)
    dst = edge_index[1].astype(jnp.int32)
    loop = jnp.arange(N, dtype=jnp.int32)
    fill_g = jnp.full((E_GAT - E - N,), _DUMMY, jnp.int32)
    src_g = jnp.concatenate([src, loop, fill_g])
    dst_g = jnp.concatenate([dst, loop, fill_g])
    fill_t = jnp.full((E_TRN - E,), _DUMMY, jnp.int32)
    src_t = jnp.concatenate([src, fill_t])
    dst_t = jnp.concatenate([dst, fill_t])

    xp = jnp.zeros((N_PAD, x.shape[1]), jnp.float32).at[:N].set(x)
    batch_p = jnp.concatenate(
        [batch.astype(jnp.int32), jnp.full((N_PAD - N,), G, jnp.int32)])

    h1 = _gat_layer(xp, src_g, dst_g, p['W1'], p['as1'], p['ad1'], p['b1'],
                    p['bn1_g'], p['bn1_b'], p['bn1_m'], p['bn1_v'],
                    8, 32, True)
    h2 = _gat_layer(h1, src_g, dst_g, p['W2'], p['as2'], p['ad2'], p['b2'],
                    p['bn2_g'], p['bn2_b'], p['bn2_m'], p['bn2_v'],
                    8, 64, True)
    h3 = _gat_layer(h2, src_g, dst_g, p['W3'], p['as3'], p['ad3'], p['b3'],
                    p['bn3_g'], p['bn3_b'], p['bn3_m'], p['bn3_v'],
                    4, 256, False)
    h4 = _trans_layer(h3, src_t, dst_t,
                      p['t1_Wq'], p['t1_bq'], p['t1_Wk'], p['t1_bk'],
                      p['t1_Wv'], p['t1_bv'], p['t1_Ws'], p['t1_bs'],
                      8, 64, True, None)
    h5 = _trans_layer(h4, src_t, dst_t,
                      p['t2_Wq'], p['t2_bq'], p['t2_Wk'], p['t2_bk'],
                      p['t2_Wv'], p['t2_bv'], p['t2_Ws'], p['t2_bs'],
                      4, 256, False, h3)

    bmin = batch_p[0::256]
    bmax = batch_p[255::256]
    mx, sm, cnt = pool_tc(h5, batch_p.reshape(N_PAD, 1), bmin, bmax)
    return final_mlp(mx, sm, cnt,
                     p['ln_g'], p['ln_b'], p['fc1_W'], p['fc1_b'],
                     p['fc2_W'], p['fc2_b'])


# speedup vs baseline: 9.6900x; 1.2332x over previous
import functools
import math

import jax
import jax.numpy as jnp
from jax import lax
from jax.experimental import pallas as pl
from jax.experimental.pallas import tpu as pltpu
from jax.experimental.pallas import tpu_sc as plsc

N = 10000
E = 160000
G = 64
N_PAD = 10240
NC, NS, LANES = 2, 16, 16
NW = NC * NS

E_GAT = 172032
E_TRN = 163840

_DUMMY = N


def _mesh():
    return plsc.VectorSubcoreMesh(core_axis_name="c", subcore_axis_name="s",
                                  num_cores=NC, num_subcores=NS)


def _leaky(x):
    return jnp.where(x >= 0, x, 0.2 * x)


def _mm_body(a_ref, w_ref, b_ref, o_ref):
    o_ref[...] = (jnp.dot(a_ref[...], w_ref[...],
                          preferred_element_type=jnp.float32) + b_ref[...])


def mm_flat(a, w, b, bm=512, bn=256):
    M, K = a.shape
    Nout = w.shape[1]
    bn = min(bn, Nout)
    return pl.pallas_call(
        _mm_body,
        grid=(M // bm, Nout // bn),
        in_specs=[pl.BlockSpec((bm, K), lambda i, j: (i, 0)),
                  pl.BlockSpec((K, bn), lambda i, j: (0, j)),
                  pl.BlockSpec((1, bn), lambda i, j: (0, j))],
        out_specs=pl.BlockSpec((bm, bn), lambda i, j: (i, j)),
        out_shape=jax.ShapeDtypeStruct((M, Nout), jnp.float32),
    )(a, w, b.reshape(1, Nout))


def _mmc_body(a_ref, w_ref, b_ref, o_ref):
    o_ref[0] = (jnp.dot(a_ref[...], w_ref[...],
                        preferred_element_type=jnp.float32) + b_ref[...])


def mm_chunk(a, w, b, bm=512):
    M, K = a.shape
    Nout = w.shape[1]
    kc = Nout // 128
    return pl.pallas_call(
        _mmc_body,
        grid=(kc, M // bm),
        in_specs=[pl.BlockSpec((bm, K), lambda j, i: (i, 0)),
                  pl.BlockSpec((K, 128), lambda j, i: (0, j)),
                  pl.BlockSpec((1, 128), lambda j, i: (0, j))],
        out_specs=pl.BlockSpec((1, bm, 128), lambda j, i: (j, i, 0)),
        out_shape=jax.ShapeDtypeStruct((kc, M, 128), jnp.float32),
    )(a, w, b.reshape(1, Nout))


def _norm_rows(num_ref, den_ref, H, C, Kc, bm):
    d = jnp.sum(den_ref[...], axis=0)[:, :H] + 1e-16
    cols = []
    for k in range(Kc):
        blk = num_ref[k]
        if C >= 128:
            hd = (k * 128) // C
            cols.append(blk / d[:, hd:hd + 1])
        else:
            nh = 128 // C
            h0 = (k * 128) // C
            dv = d[:, h0:h0 + nh]
            div = jnp.repeat(dv, C, axis=1)
            cols.append(blk / div)
    return jnp.concatenate(cols, axis=1)


def gat_post(num, den, b, bn_g, bn_b, bn_m, bn_v, H, C, concat, bm=256):
    Kc, M, _ = num.shape
    Dout = H * C if concat else C
    pp = jnp.stack([b, bn_g, bn_b, bn_m, bn_v])

    def body(num_ref, den_ref, pp_ref, o_ref):
        out = _norm_rows(num_ref, den_ref, H, C, Kc, bm)
        if not concat:
            out = out.reshape(bm, H, C).mean(axis=1)
        out = out + pp_ref[0]
        out = (out - pp_ref[3]) / jnp.sqrt(pp_ref[4] + 1e-5) * pp_ref[1] \
            + pp_ref[2]
        o_ref[...] = _leaky(out)

    return pl.pallas_call(
        body,
        grid=(M // bm,),
        in_specs=[pl.BlockSpec((Kc, bm, 128), lambda i: (0, i, 0)),
                  pl.BlockSpec((NC, bm, 128), lambda i: (0, i, 0)),
                  pl.BlockSpec((5, Dout), lambda i: (0, 0))],
        out_specs=pl.BlockSpec((bm, Dout), lambda i: (i, 0)),
        out_shape=jax.ShapeDtypeStruct((M, Dout), jnp.float32),
    )(num, den, pp)


def trans_post(num, den, skip, identity, H, C, concat, bm=256):
    Kc, M, _ = num.shape
    Dout = H * C if concat else C
    have_id = identity is not None

    def body(*refs):
        if have_id:
            num_ref, den_ref, skip_ref, id_ref, o_ref = refs
        else:
            num_ref, den_ref, skip_ref, o_ref = refs
        out = _norm_rows(num_ref, den_ref, H, C, Kc, bm)
        if not concat:
            out = out.reshape(bm, H, C).mean(axis=1)
        out = _leaky(out + skip_ref[...])
        if have_id:
            out = out + id_ref[...]
        o_ref[...] = out

    in_specs = [pl.BlockSpec((Kc, bm, 128), lambda i: (0, i, 0)),
                pl.BlockSpec((NC, bm, 128), lambda i: (0, i, 0)),
                pl.BlockSpec((bm, Dout), lambda i: (i, 0))]
    args = [num, den, skip]
    if have_id:
        in_specs.append(pl.BlockSpec((bm, Dout), lambda i: (i, 0)))
        args.append(identity)
    return pl.pallas_call(
        body,
        grid=(M // bm,),
        in_specs=in_specs,
        out_specs=pl.BlockSpec((bm, Dout), lambda i: (i, 0)),
        out_shape=jax.ShapeDtypeStruct((M, Dout), jnp.float32),
    )(*args)


def pool_tc(h, batch2d, bmin, bmax, bm=256):
    M, D = h.shape
    nblk = M // bm

    def body(bmin_ref, bmax_ref, h_ref, b_ref, mx_ref, sm_ref, ct_ref):
        i = pl.program_id(0)
        g = pl.program_id(1)

        @pl.when((i == 0) & (g == 0))
        def _init():
            mx_ref[...] = jnp.full((G, D), -1e30, jnp.float32)
            sm_ref[...] = jnp.zeros((G, D), jnp.float32)
            ct_ref[...] = jnp.zeros((G, 128), jnp.float32)

        @pl.when((g >= bmin_ref[i]) & (g <= bmax_ref[i]))
        def _acc():
            mask = b_ref[...] == g
            hb = h_ref[...]
            mxv = jnp.max(jnp.where(mask, hb, -1e30), axis=0, keepdims=True)
            smv = jnp.sum(jnp.where(mask, hb, 0.0), axis=0, keepdims=True)
            ctv = jnp.sum(mask.astype(jnp.float32))
            mx_ref[pl.ds(g, 1), :] = jnp.maximum(mx_ref[pl.ds(g, 1), :], mxv)
            sm_ref[pl.ds(g, 1), :] = sm_ref[pl.ds(g, 1), :] + smv
            ct_ref[pl.ds(g, 1), :] = ct_ref[pl.ds(g, 1), :] + ctv

    grid_spec = pltpu.PrefetchScalarGridSpec(
        num_scalar_prefetch=2,
        grid=(nblk, G),
        in_specs=[pl.BlockSpec((bm, D), lambda i, g, *_: (i, 0)),
                  pl.BlockSpec((bm, 1), lambda i, g, *_: (i, 0))],
        out_specs=[pl.BlockSpec((G, D), lambda i, g, *_: (0, 0)),
                   pl.BlockSpec((G, D), lambda i, g, *_: (0, 0)),
                   pl.BlockSpec((G, 128), lambda i, g, *_: (0, 0))])
    return pl.pallas_call(
        body, grid_spec=grid_spec,
        out_shape=[jax.ShapeDtypeStruct((G, D), jnp.float32),
                   jax.ShapeDtypeStruct((G, D), jnp.float32),
                   jax.ShapeDtypeStruct((G, 128), jnp.float32)],
    )(bmin, bmax, h, batch2d)


def final_mlp(mx, sm, cnt, ln_g, ln_b, w1, b1, w2, b2):
    D = mx.shape[-1]
    H2 = w2.shape[1]

    def body(mx_ref, sm_ref, cnt_ref, lng_ref, lnb_ref,
             w1_ref, b1_ref, w2_ref, b2_ref, o_ref):
        c = cnt_ref[:, 0:1]
        meanp = sm_ref[...] / jnp.maximum(c, 1.0)
        maxp = jnp.where(c > 0, mx_ref[...], 0.0)
        z = jnp.concatenate([maxp, meanp], axis=1)
        mu = jnp.mean(z, axis=-1, keepdims=True)
        var = jnp.mean((z - mu) ** 2, axis=-1, keepdims=True)
        z = (z - mu) / jnp.sqrt(var + 1e-5) * lng_ref[...] + lnb_ref[...]
        z = _leaky(jnp.dot(z, w1_ref[...],
                           preferred_element_type=jnp.float32) + b1_ref[...])
        z = jnp.dot(z, w2_ref[...],
                    preferred_element_type=jnp.float32) + b2_ref[...]
        o_ref[...] = jax.nn.sigmoid(z)

    return pl.pallas_call(
        body,
        out_shape=jax.ShapeDtypeStruct((G, H2), jnp.float32),
    )(mx, sm, cnt, ln_g.reshape(1, 2 * D), ln_b.reshape(1, 2 * D),
      w1, b1.reshape(1, w1.shape[1]), w2, b2.reshape(1, H2))


def _barrier():
    plsc.subcore_barrier()


def _axis_ids():
    return lax.axis_index("c"), lax.axis_index("s")


def _vperm(v, idx):
    return lax.gather(
        v, idx.reshape(16, 1),
        lax.GatherDimensionNumbers(offset_dims=(), collapsed_slice_dims=(0,),
                                   start_index_map=(0,)),
        (1,), mode=lax.GatherScatterMode.PROMISE_IN_BOUNDS)


def _lane_sum(v):
    iota = lax.broadcasted_iota(jnp.int32, (16,), 0)
    for sh in (8, 4, 2, 1):
        v = v + _vperm(v, iota ^ sh)
    return v


def _gather_rows(tab_h, idx_ref, out_ref, sem):
    pltpu.async_copy(tab_h.at[idx_ref], out_ref, sem).wait()


def _copy_start(src, dst, sem):
    return pltpu.async_copy(src, dst, sem)


def _gather_rows2(tab1, idx1, out1, sem1, tab2, idx2, out2, sem2):
    d1 = pltpu.async_copy(tab1.at[idx1], out1, sem1)
    d2 = pltpu.async_copy(tab2.at[idx2], out2, sem2)
    d1.wait()
    d2.wait()


def _scatter_add_rows(src_ref, base_ref, idx_ref):
    pltpu.sync_copy(src_ref, base_ref.at[idx_ref], add=True)


def _zero_vec_buf(buf, rows):
    def zi(i, _):
        buf[i] = jnp.zeros((16,), jnp.float32)
        return 0
    lax.fori_loop(0, rows, zi, 0, unroll=False)


@functools.lru_cache(maxsize=None)
def _make_gat_w(e_pad, H):
    epw = e_pad // NW
    B = 64
    nblk = epw // B
    zsl = N_PAD // NS

    @functools.partial(
        pl.kernel, mesh=_mesh(),
        out_type=[jax.ShapeDtypeStruct((e_pad, 16), jnp.float32),
                  jax.ShapeDtypeStruct((NC, N_PAD, 128), jnp.float32)],
        scratch_types=[
            pltpu.VMEM((64,), jnp.int32),
            pltpu.VMEM((64,), jnp.int32),
            pltpu.VMEM((64, 128), jnp.float32),
            pltpu.VMEM((64, 128), jnp.float32),
            pltpu.VMEM((64, 16), jnp.float32),
            pltpu.VMEM((64, 128), jnp.float32),
            pltpu.VMEM((16, 128), jnp.float32),
            pltpu.VMEM_SHARED((N_PAD, 128), jnp.float32),
            pltpu.SemaphoreType.DMA,
            pltpu.SemaphoreType.DMA,
        ])
    def k(src_h, dst_h, sc_h, w_h, den_h,
          srcv, dstv, ur, vr, wb, wwide, zb, dacc, sem1, sem2):
        cid, sid = _axis_ids()
        wid = sid * NC + cid

        def zrow(i, _):
            for j in range(8):
                zb[i, pl.ds(j * 16, 16)] = jnp.zeros((16,), jnp.float32)
            return 0
        lax.fori_loop(0, 16, zrow, 0, unroll=False)

        def zwide(i, _):
            for j in range(8):
                wwide[i, pl.ds(j * 16, 16)] = jnp.zeros((16,), jnp.float32)
            return 0
        lax.fori_loop(0, 64, zwide, 0, unroll=False)

        def zcp(zi, _):
            pltpu.sync_copy(zb, dacc.at[pl.ds(sid * zsl + zi * 16, 16)])
            return 0
        lax.fori_loop(0, zsl // 16, zcp, 0, unroll=False)
        _barrier()
        base0 = wid * epw

        def blk(bi, _):
            base = base0 + bi * B
            pltpu.sync_copy(src_h.at[pl.ds(base, B)], srcv)
            pltpu.sync_copy(dst_h.at[pl.ds(base, B)], dstv)
            _gather_rows2(sc_h, srcv, ur, sem1, sc_h, dstv, vr, sem2)

            def per_edge(e, _):
                a = ur[e, pl.ds(0, 16)] + vr[e, pl.ds(16, 16)]
                w = jnp.exp(jnp.where(a >= 0, a, 0.2 * a))
                wb[e] = w
                wwide[e, pl.ds(0, 16)] = w
                return 0
            lax.fori_loop(0, B, per_edge, 0, unroll=False)
            pltpu.sync_copy(wb, w_h.at[pl.ds(base, B)])
            _scatter_add_rows(wwide, dacc, dstv)
            return 0
        lax.fori_loop(0, nblk, blk, 0, unroll=False)
        _barrier()
        pltpu.sync_copy(dacc.at[pl.ds(sid * zsl, zsl)],
                        den_h.at[cid, pl.ds(sid * zsl, zsl)])

    return k


@functools.lru_cache(maxsize=None)
def _make_trans_w(e_pad, H, C, B):
    D = H * C
    epw = e_pad // NW
    nblk = epw // B
    zsl = N_PAD // NS
    scale = 1.0 / math.sqrt(float(C))

    @functools.partial(
        pl.kernel, mesh=_mesh(),
        out_type=[jax.ShapeDtypeStruct((e_pad, 16), jnp.float32),
                  jax.ShapeDtypeStruct((NC, N_PAD, 128), jnp.float32)],
        scratch_types=[
            pltpu.VMEM((B,), jnp.int32),
            pltpu.VMEM((B,), jnp.int32),
            pltpu.VMEM((B, D), jnp.float32),
            pltpu.VMEM((B, D), jnp.float32),
            pltpu.VMEM((B, 16), jnp.float32),
            pltpu.VMEM((B, 128), jnp.float32),
            pltpu.VMEM((16, 128), jnp.float32),
            pltpu.VMEM_SHARED((N_PAD, 128), jnp.float32),
            pltpu.SemaphoreType.DMA,
            pltpu.SemaphoreType.DMA,
        ])
    def k(src_h, dst_h, q_h, k_h, w_h, den_h,
          srcv, dstv, qr, kr, wb, wwide, zb, dacc, sem1, sem2):
        cid, sid = _axis_ids()
        wid = sid * NC + cid
        iota = lax.broadcasted_iota(jnp.int32, (16,), 0)

        def zrow(i, _):
            for j in range(8):
                zb[i, pl.ds(j * 16, 16)] = jnp.zeros((16,), jnp.float32)
            return 0
        lax.fori_loop(0, 16, zrow, 0, unroll=False)

        def zwide(i, _):
            for j in range(8):
                wwide[i, pl.ds(j * 16, 16)] = jnp.zeros((16,), jnp.float32)
            return 0
        lax.fori_loop(0, B, zwide, 0, unroll=False)

        def zcp(zi, _):
            pltpu.sync_copy(zb, dacc.at[pl.ds(sid * zsl + zi * 16, 16)])
            return 0
        lax.fori_loop(0, zsl // 16, zcp, 0, unroll=False)
        _barrier()
        base0 = wid * epw

        def blk(bi, _):
            base = base0 + bi * B
            pltpu.sync_copy(src_h.at[pl.ds(base, B)], srcv)
            pltpu.sync_copy(dst_h.at[pl.ds(base, B)], dstv)
            _gather_rows2(q_h, dstv, qr, sem1, k_h, srcv, kr, sem2)

            def per_edge(e, _):
                w = jnp.zeros((16,), jnp.float32)
                for hd in range(H):
                    acc = jnp.zeros((16,), jnp.float32)
                    for j in range(C // 16):
                        off = hd * C + j * 16
                        acc = acc + qr[e, pl.ds(off, 16)] * kr[e, pl.ds(off, 16)]
                    t = _lane_sum(acc) * scale
                    w = jnp.where(iota == hd, t, w)
                w = jnp.exp(jnp.where(iota < H, w, jnp.zeros((16,), jnp.float32)))
                w = jnp.where(iota < H, w, jnp.zeros((16,), jnp.float32))
                wb[e] = w
                wwide[e, pl.ds(0, 16)] = w
                return 0
            lax.fori_loop(0, B, per_edge, 0, unroll=False)
            pltpu.sync_copy(wb, w_h.at[pl.ds(base, B)])
            _scatter_add_rows(wwide, dacc, dstv)
            return 0
        lax.fori_loop(0, nblk, blk, 0, unroll=False)
        _barrier()
        pltpu.sync_copy(dacc.at[pl.ds(sid * zsl, zsl)],
                        den_h.at[cid, pl.ds(sid * zsl, zsl)])

    return k


@functools.lru_cache(maxsize=None)
def _make_agg(e_pad, K, C):
    KPC = K // NC
    epw = e_pad // NS
    B = 64
    nblk = epw // B
    zsl = N_PAD // NS
    logc = int(math.log2(C))

    @functools.partial(
        pl.kernel, mesh=_mesh(),
        out_type=jax.ShapeDtypeStruct((K, N_PAD, 128), jnp.float32),
        scratch_types=[
            pltpu.VMEM((B,), jnp.int32),
            pltpu.VMEM((B,), jnp.int32),
            pltpu.VMEM((B,), jnp.int32),
            pltpu.VMEM((B, 128), jnp.float32),
            pltpu.VMEM((B, 16), jnp.float32),
            pltpu.VMEM((16, 128), jnp.float32),
            pltpu.VMEM_SHARED((N_PAD, 128), jnp.float32),
            pltpu.SemaphoreType.DMA,
            pltpu.SemaphoreType.DMA,
            pltpu.SemaphoreType.DMA,
            pltpu.SemaphoreType.DMA,
        ])
    def k(src_h, dst_h, w_h, v_h, out_h,
          srcv, dstv, idxv, rows, wb, zb, acc, sem, semA, semB, semC):
        cid, sid = _axis_ids()

        def zrow(i, _):
            for j in range(8):
                zb[i, pl.ds(j * 16, 16)] = jnp.zeros((16,), jnp.float32)
            return 0
        lax.fori_loop(0, 16, zrow, 0, unroll=False)

        for kk in range(KPC):
            kchunk = cid * KPC + kk
            kbase = kchunk * N_PAD

            def zcp(zi, _):
                pltpu.sync_copy(zb, acc.at[pl.ds(sid * zsl + zi * 16, 16)])
                return 0
            lax.fori_loop(0, zsl // 16, zcp, 0, unroll=False)
            _barrier()

            def blk(bi, _):
                base = sid * epw + bi * B
                d1 = _copy_start(src_h.at[pl.ds(base, B)], srcv, semA)
                d2 = _copy_start(dst_h.at[pl.ds(base, B)], dstv, semB)
                d3 = _copy_start(w_h.at[pl.ds(base, B)], wb, semC)
                d1.wait()

                def mkidx(i, _):
                    idxv[pl.ds(i * 16, 16)] = srcv[pl.ds(i * 16, 16)] + kbase
                    return 0
                lax.fori_loop(0, B // 16, mkidx, 0, unroll=True)
                _gather_rows(v_h, idxv, rows, sem)
                d2.wait()
                d3.wait()

                def per_edge(e, _):
                    wv = wb[e]
                    for j in range(8):
                        hdj = (kchunk * 128 + j * 16) >> logc
                        m = _vperm(wv, jnp.full((16,), hdj, jnp.int32))
                        rows[e, pl.ds(j * 16, 16)] = rows[e, pl.ds(j * 16, 16)] * m
                    return 0
                lax.fori_loop(0, B, per_edge, 0, unroll=False)
                _scatter_add_rows(rows, acc, dstv)
                return 0
            lax.fori_loop(0, nblk, blk, 0, unroll=False)
            _barrier()
            pltpu.sync_copy(acc.at[pl.ds(sid * zsl, zsl)],
                            out_h.at[kchunk, pl.ds(sid * zsl, zsl)])
            _barrier()

    return k


def _score_weights(W, a_s, a_d, H, C):
    Din = W.shape[0]
    ws = (W.reshape(Din, H, C) * a_s[None]).sum(-1)
    wd = (W.reshape(Din, H, C) * a_d[None]).sum(-1)
    Wsc = jnp.zeros((Din, 128), jnp.float32)
    Wsc = Wsc.at[:, 0:H].set(ws).at[:, 16:16 + H].set(wd)
    return Wsc


def _gat_layer(h, src, dst, W, a_s, a_d, b, bn_g, bn_b, bn_m, bn_v,
               H, C, concat):
    Kc = (H * C) // 128
    hp = mm_chunk(h, W, jnp.zeros((H * C,), jnp.float32))
    sc = mm_flat(h, _score_weights(W, a_s, a_d, H, C),
                 jnp.zeros((128,), jnp.float32), bn=128)
    w, den = _make_gat_w(E_GAT, H)(src, dst, sc)
    num = _make_agg(E_GAT, Kc, C)(src, dst, w, hp.reshape(Kc * N_PAD, 128))
    return gat_post(num, den, b, bn_g, bn_b, bn_m, bn_v, H, C, concat)


def _trans_layer(h, src, dst, Wq, bq, Wk, bk, Wv, bv, Ws, bs,
                 H, C, concat, identity):
    Kc = (H * C) // 128
    q = mm_flat(h, Wq, bq)
    kt = mm_flat(h, Wk, bk)
    v = mm_chunk(h, Wv, bv)
    skip = mm_flat(h, Ws, bs)
    B = 16
    w, den = _make_trans_w(E_TRN, H, C, B)(src, dst, q, kt)
    num = _make_agg(E_TRN, Kc, C)(src, dst, w, v.reshape(Kc * N_PAD, 128))
    return trans_post(num, den, skip, identity, H, C, concat)


def kernel(x, edge_index, batch, params):
    p = params
    src = edge_index[0].astype(jnp.int32)
    dst = edge_index[1].astype(jnp.int32)
    loop = jnp.arange(N, dtype=jnp.int32)
    fill_g = jnp.full((E_GAT - E - N,), _DUMMY, jnp.int32)
    src_g = jnp.concatenate([src, loop, fill_g])
    dst_g = jnp.concatenate([dst, loop, fill_g])
    fill_t = jnp.full((E_TRN - E,), _DUMMY, jnp.int32)
    src_t = jnp.concatenate([src, fill_t])
    dst_t = jnp.concatenate([dst, fill_t])

    xp = jnp.zeros((N_PAD, x.shape[1]), jnp.float32).at[:N].set(x)
    batch_p = jnp.concatenate(
        [batch.astype(jnp.int32), jnp.full((N_PAD - N,), G, jnp.int32)])

    h1 = _gat_layer(xp, src_g, dst_g, p['W1'], p['as1'], p['ad1'], p['b1'],
                    p['bn1_g'], p['bn1_b'], p['bn1_m'], p['bn1_v'],
                    8, 32, True)
    h2 = _gat_layer(h1, src_g, dst_g, p['W2'], p['as2'], p['ad2'], p['b2'],
                    p['bn2_g'], p['bn2_b'], p['bn2_m'], p['bn2_v'],
                    8, 64, True)
    h3 = _gat_layer(h2, src_g, dst_g, p['W3'], p['as3'], p['ad3'], p['b3'],
                    p['bn3_g'], p['bn3_b'], p['bn3_m'], p['bn3_v'],
                    4, 256, False)
    h4 = _trans_layer(h3, src_t, dst_t,
                      p['t1_Wq'], p['t1_bq'], p['t1_Wk'], p['t1_bk'],
                      p['t1_Wv'], p['t1_bv'], p['t1_Ws'], p['t1_bs'],
                      8, 64, True, None)
    h5 = _trans_layer(h4, src_t, dst_t,
                      p['t2_Wq'], p['t2_bq'], p['t2_Wk'], p['t2_bk'],
                      p['t2_Wv'], p['t2_bv'], p['t2_Ws'], p['t2_bs'],
                      4, 256, False, h3)

    bmin = batch_p[0::256]
    bmax = batch_p[255::256]
    mx, sm, cnt = pool_tc(h5, batch_p.reshape(N_PAD, 1), bmin, bmax)
    return final_mlp(mx, sm, cnt,
                     p['ln_g'], p['ln_b'], p['fc1_W'], p['fc1_b'],
                     p['fc2_W'], p['fc2_b'])


# speedup vs baseline: 10.7651x; 1.1109x over previous
import functools
import math

import jax
import jax.numpy as jnp
from jax import lax
from jax.experimental import pallas as pl
from jax.experimental.pallas import tpu as pltpu
from jax.experimental.pallas import tpu_sc as plsc

N = 10000
E = 160000
G = 64
N_PAD = 10240
NC, NS, LANES = 2, 16, 16
NW = NC * NS

E_GAT = 172032
E_TRN = 163840

_DUMMY = N


def _mesh():
    return plsc.VectorSubcoreMesh(core_axis_name="c", subcore_axis_name="s",
                                  num_cores=NC, num_subcores=NS)


def _leaky(x):
    return jnp.where(x >= 0, x, 0.2 * x)


def _mm_body(a_ref, w_ref, b_ref, o_ref):
    o_ref[...] = (jnp.dot(a_ref[...], w_ref[...],
                          preferred_element_type=jnp.float32) + b_ref[...])


def mm_flat(a, w, b, bm=512, bn=256):
    M, K = a.shape
    Nout = w.shape[1]
    bn = min(bn, Nout)
    return pl.pallas_call(
        _mm_body,
        grid=(M // bm, Nout // bn),
        in_specs=[pl.BlockSpec((bm, K), lambda i, j: (i, 0)),
                  pl.BlockSpec((K, bn), lambda i, j: (0, j)),
                  pl.BlockSpec((1, bn), lambda i, j: (0, j))],
        out_specs=pl.BlockSpec((bm, bn), lambda i, j: (i, j)),
        out_shape=jax.ShapeDtypeStruct((M, Nout), jnp.float32),
    )(a, w, b.reshape(1, Nout))


def _mmc_body(a_ref, w_ref, b_ref, o_ref):
    o_ref[0] = (jnp.dot(a_ref[...], w_ref[...],
                        preferred_element_type=jnp.float32) + b_ref[...])


def mm_chunk(a, w, b, bm=512):
    M, K = a.shape
    Nout = w.shape[1]
    kc = Nout // 128
    return pl.pallas_call(
        _mmc_body,
        grid=(kc, M // bm),
        in_specs=[pl.BlockSpec((bm, K), lambda j, i: (i, 0)),
                  pl.BlockSpec((K, 128), lambda j, i: (0, j)),
                  pl.BlockSpec((1, 128), lambda j, i: (0, j))],
        out_specs=pl.BlockSpec((1, bm, 128), lambda j, i: (j, i, 0)),
        out_shape=jax.ShapeDtypeStruct((kc, M, 128), jnp.float32),
    )(a, w, b.reshape(1, Nout))


def _norm_rows(num_ref, den_ref, H, C, Kc, bm):
    d = jnp.sum(den_ref[...], axis=0)[:, :H] + 1e-16
    cols = []
    for k in range(Kc):
        blk = num_ref[k]
        if C >= 128:
            hd = (k * 128) // C
            cols.append(blk / d[:, hd:hd + 1])
        else:
            nh = 128 // C
            h0 = (k * 128) // C
            dv = d[:, h0:h0 + nh]
            div = jnp.repeat(dv, C, axis=1)
            cols.append(blk / div)
    return jnp.concatenate(cols, axis=1)


def gat_post(num, den, b, bn_g, bn_b, bn_m, bn_v, H, C, concat, bm=256):
    Kc, M, _ = num.shape
    Dout = H * C if concat else C
    pp = jnp.stack([b, bn_g, bn_b, bn_m, bn_v])

    def body(num_ref, den_ref, pp_ref, o_ref):
        out = _norm_rows(num_ref, den_ref, H, C, Kc, bm)
        if not concat:
            out = out.reshape(bm, H, C).mean(axis=1)
        out = out + pp_ref[0]
        out = (out - pp_ref[3]) / jnp.sqrt(pp_ref[4] + 1e-5) * pp_ref[1] \
            + pp_ref[2]
        o_ref[...] = _leaky(out)

    return pl.pallas_call(
        body,
        grid=(M // bm,),
        in_specs=[pl.BlockSpec((Kc, bm, 128), lambda i: (0, i, 0)),
                  pl.BlockSpec((NC, bm, 128), lambda i: (0, i, 0)),
                  pl.BlockSpec((5, Dout), lambda i: (0, 0))],
        out_specs=pl.BlockSpec((bm, Dout), lambda i: (i, 0)),
        out_shape=jax.ShapeDtypeStruct((M, Dout), jnp.float32),
    )(num, den, pp)


def trans_post(num, den, skip, identity, H, C, concat, bm=256):
    Kc, M, _ = num.shape
    Dout = H * C if concat else C
    have_id = identity is not None

    def body(*refs):
        if have_id:
            num_ref, den_ref, skip_ref, id_ref, o_ref = refs
        else:
            num_ref, den_ref, skip_ref, o_ref = refs
        out = _norm_rows(num_ref, den_ref, H, C, Kc, bm)
        if not concat:
            out = out.reshape(bm, H, C).mean(axis=1)
        out = _leaky(out + skip_ref[...])
        if have_id:
            out = out + id_ref[...]
        o_ref[...] = out

    in_specs = [pl.BlockSpec((Kc, bm, 128), lambda i: (0, i, 0)),
                pl.BlockSpec((NC, bm, 128), lambda i: (0, i, 0)),
                pl.BlockSpec((bm, Dout), lambda i: (i, 0))]
    args = [num, den, skip]
    if have_id:
        in_specs.append(pl.BlockSpec((bm, Dout), lambda i: (i, 0)))
        args.append(identity)
    return pl.pallas_call(
        body,
        grid=(M // bm,),
        in_specs=in_specs,
        out_specs=pl.BlockSpec((bm, Dout), lambda i: (i, 0)),
        out_shape=jax.ShapeDtypeStruct((M, Dout), jnp.float32),
    )(*args)


def pool_tc(h, batch2d, bmin, bmax, bm=256):
    M, D = h.shape
    nblk = M // bm

    def body(bmin_ref, bmax_ref, h_ref, b_ref, mx_ref, sm_ref, ct_ref):
        i = pl.program_id(0)
        g = pl.program_id(1)

        @pl.when((i == 0) & (g == 0))
        def _init():
            mx_ref[...] = jnp.full((G, D), -1e30, jnp.float32)
            sm_ref[...] = jnp.zeros((G, D), jnp.float32)
            ct_ref[...] = jnp.zeros((G, 128), jnp.float32)

        @pl.when((g >= bmin_ref[i]) & (g <= bmax_ref[i]))
        def _acc():
            mask = b_ref[...] == g
            hb = h_ref[...]
            mxv = jnp.max(jnp.where(mask, hb, -1e30), axis=0, keepdims=True)
            smv = jnp.sum(jnp.where(mask, hb, 0.0), axis=0, keepdims=True)
            ctv = jnp.sum(mask.astype(jnp.float32))
            mx_ref[pl.ds(g, 1), :] = jnp.maximum(mx_ref[pl.ds(g, 1), :], mxv)
            sm_ref[pl.ds(g, 1), :] = sm_ref[pl.ds(g, 1), :] + smv
            ct_ref[pl.ds(g, 1), :] = ct_ref[pl.ds(g, 1), :] + ctv

    grid_spec = pltpu.PrefetchScalarGridSpec(
        num_scalar_prefetch=2,
        grid=(nblk, G),
        in_specs=[pl.BlockSpec((bm, D), lambda i, g, *_: (i, 0)),
                  pl.BlockSpec((bm, 1), lambda i, g, *_: (i, 0))],
        out_specs=[pl.BlockSpec((G, D), lambda i, g, *_: (0, 0)),
                   pl.BlockSpec((G, D), lambda i, g, *_: (0, 0)),
                   pl.BlockSpec((G, 128), lambda i, g, *_: (0, 0))])
    return pl.pallas_call(
        body, grid_spec=grid_spec,
        out_shape=[jax.ShapeDtypeStruct((G, D), jnp.float32),
                   jax.ShapeDtypeStruct((G, D), jnp.float32),
                   jax.ShapeDtypeStruct((G, 128), jnp.float32)],
    )(bmin, bmax, h, batch2d)


def final_mlp(mx, sm, cnt, ln_g, ln_b, w1, b1, w2, b2):
    D = mx.shape[-1]
    H2 = w2.shape[1]

    def body(mx_ref, sm_ref, cnt_ref, lng_ref, lnb_ref,
             w1_ref, b1_ref, w2_ref, b2_ref, o_ref):
        c = cnt_ref[:, 0:1]
        meanp = sm_ref[...] / jnp.maximum(c, 1.0)
        maxp = jnp.where(c > 0, mx_ref[...], 0.0)
        z = jnp.concatenate([maxp, meanp], axis=1)
        mu = jnp.mean(z, axis=-1, keepdims=True)
        var = jnp.mean((z - mu) ** 2, axis=-1, keepdims=True)
        z = (z - mu) / jnp.sqrt(var + 1e-5) * lng_ref[...] + lnb_ref[...]
        z = _leaky(jnp.dot(z, w1_ref[...],
                           preferred_element_type=jnp.float32) + b1_ref[...])
        z = jnp.dot(z, w2_ref[...],
                    preferred_element_type=jnp.float32) + b2_ref[...]
        o_ref[...] = jax.nn.sigmoid(z)

    return pl.pallas_call(
        body,
        out_shape=jax.ShapeDtypeStruct((G, H2), jnp.float32),
    )(mx, sm, cnt, ln_g.reshape(1, 2 * D), ln_b.reshape(1, 2 * D),
      w1, b1.reshape(1, w1.shape[1]), w2, b2.reshape(1, H2))


def _barrier():
    plsc.subcore_barrier()


def _axis_ids():
    return lax.axis_index("c"), lax.axis_index("s")


def _vperm(v, idx):
    return lax.gather(
        v, idx.reshape(16, 1),
        lax.GatherDimensionNumbers(offset_dims=(), collapsed_slice_dims=(0,),
                                   start_index_map=(0,)),
        (1,), mode=lax.GatherScatterMode.PROMISE_IN_BOUNDS)


def _lane_sum(v):
    iota = lax.broadcasted_iota(jnp.int32, (16,), 0)
    for sh in (8, 4, 2, 1):
        v = v + _vperm(v, iota ^ sh)
    return v


def _gather_rows(tab_h, idx_ref, out_ref, sem):
    pltpu.async_copy(tab_h.at[idx_ref], out_ref, sem).wait()


def _copy_start(src, dst, sem):
    return pltpu.async_copy(src, dst, sem)


def _gather_rows2(tab1, idx1, out1, sem1, tab2, idx2, out2, sem2):
    d1 = pltpu.async_copy(tab1.at[idx1], out1, sem1)
    d2 = pltpu.async_copy(tab2.at[idx2], out2, sem2)
    d1.wait()
    d2.wait()


def _scatter_add_rows(src_ref, base_ref, idx_ref):
    pltpu.sync_copy(src_ref, base_ref.at[idx_ref], add=True)


def _zero_vec_buf(buf, rows):
    def zi(i, _):
        buf[i] = jnp.zeros((16,), jnp.float32)
        return 0
    lax.fori_loop(0, rows, zi, 0, unroll=False)


@functools.lru_cache(maxsize=None)
def _make_gat_w(e_pad, H):
    epw = e_pad // NW
    B = 64
    nblk = epw // B
    zsl = N_PAD // NS

    @functools.partial(
        pl.kernel, mesh=_mesh(),
        out_type=[jax.ShapeDtypeStruct((e_pad, 16), jnp.float32),
                  jax.ShapeDtypeStruct((NC, N_PAD, 128), jnp.float32)],
        scratch_types=[
            pltpu.VMEM((64,), jnp.int32),
            pltpu.VMEM((64,), jnp.int32),
            pltpu.VMEM((64, 128), jnp.float32),
            pltpu.VMEM((64, 128), jnp.float32),
            pltpu.VMEM((64, 16), jnp.float32),
            pltpu.VMEM((64, 128), jnp.float32),
            pltpu.VMEM((16, 128), jnp.float32),
            pltpu.VMEM_SHARED((N_PAD, 128), jnp.float32),
            pltpu.SemaphoreType.DMA,
            pltpu.SemaphoreType.DMA,
        ])
    def k(src_h, dst_h, sc_h, w_h, den_h,
          srcv, dstv, ur, vr, wb, wwide, zb, dacc, sem1, sem2):
        cid, sid = _axis_ids()
        wid = sid * NC + cid

        def zrow(i, _):
            for j in range(8):
                zb[i, pl.ds(j * 16, 16)] = jnp.zeros((16,), jnp.float32)
            return 0
        lax.fori_loop(0, 16, zrow, 0, unroll=False)

        def zwide(i, _):
            for j in range(8):
                wwide[i, pl.ds(j * 16, 16)] = jnp.zeros((16,), jnp.float32)
            return 0
        lax.fori_loop(0, 64, zwide, 0, unroll=False)

        def zcp(zi, _):
            pltpu.sync_copy(zb, dacc.at[pl.ds(sid * zsl + zi * 16, 16)])
            return 0
        lax.fori_loop(0, zsl // 16, zcp, 0, unroll=False)
        _barrier()
        base0 = wid * epw

        def blk(bi, _):
            base = base0 + bi * B
            pltpu.sync_copy(src_h.at[pl.ds(base, B)], srcv)
            pltpu.sync_copy(dst_h.at[pl.ds(base, B)], dstv)
            _gather_rows2(sc_h, srcv, ur, sem1, sc_h, dstv, vr, sem2)

            def per_edge(e, _):
                a = ur[e, pl.ds(0, 16)] + vr[e, pl.ds(16, 16)]
                w = jnp.exp(jnp.where(a >= 0, a, 0.2 * a))
                wb[e] = w
                wwide[e, pl.ds(0, 16)] = w
                return 0
            lax.fori_loop(0, B, per_edge, 0, unroll=False)
            pltpu.sync_copy(wb, w_h.at[pl.ds(base, B)])
            _scatter_add_rows(wwide, dacc, dstv)
            return 0
        lax.fori_loop(0, nblk, blk, 0, unroll=False)
        _barrier()
        pltpu.sync_copy(dacc.at[pl.ds(sid * zsl, zsl)],
                        den_h.at[cid, pl.ds(sid * zsl, zsl)])

    return k


@functools.lru_cache(maxsize=None)
def _make_trans_w(e_pad, H, C, B):
    D = H * C
    epw = e_pad // NW
    nblk = epw // B
    zsl = N_PAD // NS
    scale = 1.0 / math.sqrt(float(C))

    @functools.partial(
        pl.kernel, mesh=_mesh(),
        out_type=[jax.ShapeDtypeStruct((e_pad, 16), jnp.float32),
                  jax.ShapeDtypeStruct((NC, N_PAD, 128), jnp.float32)],
        scratch_types=[
            pltpu.VMEM((B,), jnp.int32),
            pltpu.VMEM((B,), jnp.int32),
            pltpu.VMEM((B, D), jnp.float32),
            pltpu.VMEM((B, D), jnp.float32),
            pltpu.VMEM((B, 16), jnp.float32),
            pltpu.VMEM((B, 128), jnp.float32),
            pltpu.VMEM((16, 128), jnp.float32),
            pltpu.VMEM_SHARED((N_PAD, 128), jnp.float32),
            pltpu.SemaphoreType.DMA,
            pltpu.SemaphoreType.DMA,
        ])
    def k(src_h, dst_h, q_h, k_h, w_h, den_h,
          srcv, dstv, qr, kr, wb, wwide, zb, dacc, sem1, sem2):
        cid, sid = _axis_ids()
        wid = sid * NC + cid
        iota = lax.broadcasted_iota(jnp.int32, (16,), 0)

        def zrow(i, _):
            for j in range(8):
                zb[i, pl.ds(j * 16, 16)] = jnp.zeros((16,), jnp.float32)
            return 0
        lax.fori_loop(0, 16, zrow, 0, unroll=False)

        def zwide(i, _):
            for j in range(8):
                wwide[i, pl.ds(j * 16, 16)] = jnp.zeros((16,), jnp.float32)
            return 0
        lax.fori_loop(0, B, zwide, 0, unroll=False)

        def zcp(zi, _):
            pltpu.sync_copy(zb, dacc.at[pl.ds(sid * zsl + zi * 16, 16)])
            return 0
        lax.fori_loop(0, zsl // 16, zcp, 0, unroll=False)
        _barrier()
        base0 = wid * epw

        def blk(bi, _):
            base = base0 + bi * B
            pltpu.sync_copy(src_h.at[pl.ds(base, B)], srcv)
            pltpu.sync_copy(dst_h.at[pl.ds(base, B)], dstv)
            _gather_rows2(q_h, dstv, qr, sem1, k_h, srcv, kr, sem2)

            def per_edge(e, _):
                w = jnp.zeros((16,), jnp.float32)
                for hd in range(H):
                    acc = jnp.zeros((16,), jnp.float32)
                    for j in range(C // 16):
                        off = hd * C + j * 16
                        acc = acc + qr[e, pl.ds(off, 16)] * kr[e, pl.ds(off, 16)]
                    t = _lane_sum(acc) * scale
                    w = jnp.where(iota == hd, t, w)
                w = jnp.exp(jnp.where(iota < H, w, jnp.zeros((16,), jnp.float32)))
                w = jnp.where(iota < H, w, jnp.zeros((16,), jnp.float32))
                wb[e] = w
                wwide[e, pl.ds(0, 16)] = w
                return 0
            lax.fori_loop(0, B, per_edge, 0, unroll=False)
            pltpu.sync_copy(wb, w_h.at[pl.ds(base, B)])
            _scatter_add_rows(wwide, dacc, dstv)
            return 0
        lax.fori_loop(0, nblk, blk, 0, unroll=False)
        _barrier()
        pltpu.sync_copy(dacc.at[pl.ds(sid * zsl, zsl)],
                        den_h.at[cid, pl.ds(sid * zsl, zsl)])

    return k


@functools.lru_cache(maxsize=None)
def _make_agg(e_pad, K, C):
    KPC = K // NC
    epw = e_pad // NS
    B = 64
    nblk = epw // B
    assert nblk % 2 == 0
    zsl = N_PAD // NS
    logc = int(math.log2(C))

    @functools.partial(
        pl.kernel, mesh=_mesh(),
        out_type=jax.ShapeDtypeStruct((K, N_PAD, 128), jnp.float32),
        scratch_types=[
            [pltpu.VMEM((B,), jnp.int32)] * 2,
            [pltpu.VMEM((B,), jnp.int32)] * 2,
            [pltpu.VMEM((B,), jnp.int32)] * 2,
            [pltpu.VMEM((B, 128), jnp.float32)] * 2,
            [pltpu.VMEM((B, 16), jnp.float32)] * 2,
            pltpu.VMEM((16, 128), jnp.float32),
            pltpu.VMEM_SHARED((N_PAD, 128), jnp.float32),
            [pltpu.SemaphoreType.DMA] * 2,
            [pltpu.SemaphoreType.DMA] * 2,
            [pltpu.SemaphoreType.DMA] * 2,
            [pltpu.SemaphoreType.DMA] * 2,
        ])
    def k(src_h, dst_h, w_h, v_h, out_h,
          srcv, dstv, idxv, rows, wb, zb, acc, semg, semA, semB, semC):
        cid, sid = _axis_ids()

        def zrow(i, _):
            for j in range(8):
                zb[i, pl.ds(j * 16, 16)] = jnp.zeros((16,), jnp.float32)
            return 0
        lax.fori_loop(0, 16, zrow, 0, unroll=False)

        for kk in range(KPC):
            kchunk = cid * KPC + kk
            kbase = kchunk * N_PAD

            def zcp(zi, _):
                pltpu.sync_copy(zb, acc.at[pl.ds(sid * zsl + zi * 16, 16)])
                return 0
            lax.fori_loop(0, zsl // 16, zcp, 0, unroll=False)
            _barrier()

            def stage(p, base):
                d1 = _copy_start(src_h.at[pl.ds(base, B)], srcv[p], semA[p])
                d2 = _copy_start(dst_h.at[pl.ds(base, B)], dstv[p], semB[p])
                d3 = _copy_start(w_h.at[pl.ds(base, B)], wb[p], semC[p])
                d1.wait()

                def mkidx(i, _):
                    idxv[p][pl.ds(i * 16, 16)] = (
                        srcv[p][pl.ds(i * 16, 16)] + kbase)
                    return 0
                lax.fori_loop(0, B // 16, mkidx, 0, unroll=True)
                dg = _copy_start(v_h.at[idxv[p]], rows[p], semg[p])
                return d2, d3, dg

            def drain(p, d2, d3, dg):
                d3.wait()
                dg.wait()

                def per_edge(e, _):
                    wv = wb[p][e]
                    for j in range(8):
                        hdj = (kchunk * 128 + j * 16) >> logc
                        m = _vperm(wv, jnp.full((16,), hdj, jnp.int32))
                        rows[p][e, pl.ds(j * 16, 16)] = (
                            rows[p][e, pl.ds(j * 16, 16)] * m)
                    return 0
                lax.fori_loop(0, B, per_edge, 0, unroll=False)
                d2.wait()
                _scatter_add_rows(rows[p], acc, dstv[p])

            def blk(pi, _):
                base = sid * epw + pi * 2 * B
                da = stage(0, base)
                db = stage(1, base + B)
                drain(0, *da)
                drain(1, *db)
                return 0
            lax.fori_loop(0, nblk // 2, blk, 0, unroll=False)
            _barrier()
            pltpu.sync_copy(acc.at[pl.ds(sid * zsl, zsl)],
                            out_h.at[kchunk, pl.ds(sid * zsl, zsl)])
            _barrier()

    return k


def _score_weights(W, a_s, a_d, H, C):
    Din = W.shape[0]
    ws = (W.reshape(Din, H, C) * a_s[None]).sum(-1)
    wd = (W.reshape(Din, H, C) * a_d[None]).sum(-1)
    Wsc = jnp.zeros((Din, 128), jnp.float32)
    Wsc = Wsc.at[:, 0:H].set(ws).at[:, 16:16 + H].set(wd)
    return Wsc


def _gat_layer(h, src, dst, W, a_s, a_d, b, bn_g, bn_b, bn_m, bn_v,
               H, C, concat):
    Kc = (H * C) // 128
    hp = mm_chunk(h, W, jnp.zeros((H * C,), jnp.float32))
    sc = mm_flat(h, _score_weights(W, a_s, a_d, H, C),
                 jnp.zeros((128,), jnp.float32), bn=128)
    w, den = _make_gat_w(E_GAT, H)(src, dst, sc)
    num = _make_agg(E_GAT, Kc, C)(src, dst, w, hp.reshape(Kc * N_PAD, 128))
    return gat_post(num, den, b, bn_g, bn_b, bn_m, bn_v, H, C, concat)


def _trans_layer(h, src, dst, Wq, bq, Wk, bk, Wv, bv, Ws, bs,
                 H, C, concat, identity):
    Kc = (H * C) // 128
    q = mm_flat(h, Wq, bq)
    kt = mm_flat(h, Wk, bk)
    v = mm_chunk(h, Wv, bv)
    skip = mm_flat(h, Ws, bs)
    B = 16
    w, den = _make_trans_w(E_TRN, H, C, B)(src, dst, q, kt)
    num = _make_agg(E_TRN, Kc, C)(src, dst, w, v.reshape(Kc * N_PAD, 128))
    return trans_post(num, den, skip, identity, H, C, concat)


def kernel(x, edge_index, batch, params):
    p = params
    src = edge_index[0].astype(jnp.int32)
    dst = edge_index[1].astype(jnp.int32)
    loop = jnp.arange(N, dtype=jnp.int32)
    fill_g = jnp.full((E_GAT - E - N,), _DUMMY, jnp.int32)
    src_g = jnp.concatenate([src, loop, fill_g])
    dst_g = jnp.concatenate([dst, loop, fill_g])
    fill_t = jnp.full((E_TRN - E,), _DUMMY, jnp.int32)
    src_t = jnp.concatenate([src, fill_t])
    dst_t = jnp.concatenate([dst, fill_t])

    xp = jnp.zeros((N_PAD, x.shape[1]), jnp.float32).at[:N].set(x)
    batch_p = jnp.concatenate(
        [batch.astype(jnp.int32), jnp.full((N_PAD - N,), G, jnp.int32)])

    h1 = _gat_layer(xp, src_g, dst_g, p['W1'], p['as1'], p['ad1'], p['b1'],
                    p['bn1_g'], p['bn1_b'], p['bn1_m'], p['bn1_v'],
                    8, 32, True)
    h2 = _gat_layer(h1, src_g, dst_g, p['W2'], p['as2'], p['ad2'], p['b2'],
                    p['bn2_g'], p['bn2_b'], p['bn2_m'], p['bn2_v'],
                    8, 64, True)
    h3 = _gat_layer(h2, src_g, dst_g, p['W3'], p['as3'], p['ad3'], p['b3'],
                    p['bn3_g'], p['bn3_b'], p['bn3_m'], p['bn3_v'],
                    4, 256, False)
    h4 = _trans_layer(h3, src_t, dst_t,
                      p['t1_Wq'], p['t1_bq'], p['t1_Wk'], p['t1_bk'],
                      p['t1_Wv'], p['t1_bv'], p['t1_Ws'], p['t1_bs'],
                      8, 64, True, None)
    h5 = _trans_layer(h4, src_t, dst_t,
                      p['t2_Wq'], p['t2_bq'], p['t2_Wk'], p['t2_bk'],
                      p['t2_Wv'], p['t2_bv'], p['t2_Ws'], p['t2_bs'],
                      4, 256, False, h3)

    bmin = batch_p[0::256]
    bmax = batch_p[255::256]
    mx, sm, cnt = pool_tc(h5, batch_p.reshape(N_PAD, 1), bmin, bmax)
    return final_mlp(mx, sm, cnt,
                     p['ln_g'], p['ln_b'], p['fc1_W'], p['fc1_b'],
                     p['fc2_W'], p['fc2_b'])


# speedup vs baseline: 11.8083x; 1.0969x over previous
import functools
import math

import jax
import jax.numpy as jnp
from jax import lax
from jax.experimental import pallas as pl
from jax.experimental.pallas import tpu as pltpu
from jax.experimental.pallas import tpu_sc as plsc

N = 10000
E = 160000
G = 64
N_PAD = 10240
NC, NS, LANES = 2, 16, 16
NW = NC * NS

E_GAT = 172032
E_TRN = 163840

_DUMMY = N


def _mesh():
    return plsc.VectorSubcoreMesh(core_axis_name="c", subcore_axis_name="s",
                                  num_cores=NC, num_subcores=NS)


def _leaky(x):
    return jnp.where(x >= 0, x, 0.2 * x)


def _mm_body(a_ref, w_ref, b_ref, o_ref):
    o_ref[...] = (jnp.dot(a_ref[...], w_ref[...],
                          preferred_element_type=jnp.float32) + b_ref[...])


def mm_flat(a, w, b, bm=512, bn=256):
    M, K = a.shape
    Nout = w.shape[1]
    bn = min(bn, Nout)
    return pl.pallas_call(
        _mm_body,
        grid=(M // bm, Nout // bn),
        in_specs=[pl.BlockSpec((bm, K), lambda i, j: (i, 0)),
                  pl.BlockSpec((K, bn), lambda i, j: (0, j)),
                  pl.BlockSpec((1, bn), lambda i, j: (0, j))],
        out_specs=pl.BlockSpec((bm, bn), lambda i, j: (i, j)),
        out_shape=jax.ShapeDtypeStruct((M, Nout), jnp.float32),
    )(a, w, b.reshape(1, Nout))


def _mmc_body(a_ref, w_ref, b_ref, o_ref):
    o_ref[0] = (jnp.dot(a_ref[...], w_ref[...],
                        preferred_element_type=jnp.float32) + b_ref[...])


def mm_chunk(a, w, b, bm=512):
    M, K = a.shape
    Nout = w.shape[1]
    kc = Nout // 128
    return pl.pallas_call(
        _mmc_body,
        grid=(kc, M // bm),
        in_specs=[pl.BlockSpec((bm, K), lambda j, i: (i, 0)),
                  pl.BlockSpec((K, 128), lambda j, i: (0, j)),
                  pl.BlockSpec((1, 128), lambda j, i: (0, j))],
        out_specs=pl.BlockSpec((1, bm, 128), lambda j, i: (j, i, 0)),
        out_shape=jax.ShapeDtypeStruct((kc, M, 128), jnp.float32),
    )(a, w, b.reshape(1, Nout))


def _norm_rows(num_ref, den_ref, H, C, Kc, bm):
    d = jnp.sum(den_ref[...], axis=0)[:, :H] + 1e-16
    cols = []
    for k in range(Kc):
        blk = num_ref[k]
        if C >= 128:
            hd = (k * 128) // C
            cols.append(blk / d[:, hd:hd + 1])
        else:
            nh = 128 // C
            h0 = (k * 128) // C
            dv = d[:, h0:h0 + nh]
            div = jnp.repeat(dv, C, axis=1)
            cols.append(blk / div)
    return jnp.concatenate(cols, axis=1)


def gat_post(num, den, b, bn_g, bn_b, bn_m, bn_v, H, C, concat, bm=256):
    Kc, M, _ = num.shape
    Dout = H * C if concat else C
    pp = jnp.stack([b, bn_g, bn_b, bn_m, bn_v])

    def body(num_ref, den_ref, pp_ref, o_ref):
        out = _norm_rows(num_ref, den_ref, H, C, Kc, bm)
        if not concat:
            out = out.reshape(bm, H, C).mean(axis=1)
        out = out + pp_ref[0]
        out = (out - pp_ref[3]) / jnp.sqrt(pp_ref[4] + 1e-5) * pp_ref[1] \
            + pp_ref[2]
        o_ref[...] = _leaky(out)

    return pl.pallas_call(
        body,
        grid=(M // bm,),
        in_specs=[pl.BlockSpec((Kc, bm, 128), lambda i: (0, i, 0)),
                  pl.BlockSpec((NC, bm, 128), lambda i: (0, i, 0)),
                  pl.BlockSpec((5, Dout), lambda i: (0, 0))],
        out_specs=pl.BlockSpec((bm, Dout), lambda i: (i, 0)),
        out_shape=jax.ShapeDtypeStruct((M, Dout), jnp.float32),
    )(num, den, pp)


def trans_post(num, den, skip, identity, H, C, concat, bm=256):
    Kc, M, _ = num.shape
    Dout = H * C if concat else C
    have_id = identity is not None

    def body(*refs):
        if have_id:
            num_ref, den_ref, skip_ref, id_ref, o_ref = refs
        else:
            num_ref, den_ref, skip_ref, o_ref = refs
        out = _norm_rows(num_ref, den_ref, H, C, Kc, bm)
        if not concat:
            out = out.reshape(bm, H, C).mean(axis=1)
        out = _leaky(out + skip_ref[...])
        if have_id:
            out = out + id_ref[...]
        o_ref[...] = out

    in_specs = [pl.BlockSpec((Kc, bm, 128), lambda i: (0, i, 0)),
                pl.BlockSpec((NC, bm, 128), lambda i: (0, i, 0)),
                pl.BlockSpec((bm, Dout), lambda i: (i, 0))]
    args = [num, den, skip]
    if have_id:
        in_specs.append(pl.BlockSpec((bm, Dout), lambda i: (i, 0)))
        args.append(identity)
    return pl.pallas_call(
        body,
        grid=(M // bm,),
        in_specs=in_specs,
        out_specs=pl.BlockSpec((bm, Dout), lambda i: (i, 0)),
        out_shape=jax.ShapeDtypeStruct((M, Dout), jnp.float32),
    )(*args)


def pool_tc(h, batch2d, bmin, bmax, bm=256):
    M, D = h.shape
    nblk = M // bm

    def body(bmin_ref, bmax_ref, h_ref, b_ref, mx_ref, sm_ref, ct_ref):
        i = pl.program_id(0)
        g = pl.program_id(1)

        @pl.when((i == 0) & (g == 0))
        def _init():
            mx_ref[...] = jnp.full((G, D), -1e30, jnp.float32)
            sm_ref[...] = jnp.zeros((G, D), jnp.float32)
            ct_ref[...] = jnp.zeros((G, 128), jnp.float32)

        @pl.when((g >= bmin_ref[i]) & (g <= bmax_ref[i]))
        def _acc():
            mask = b_ref[...] == g
            hb = h_ref[...]
            mxv = jnp.max(jnp.where(mask, hb, -1e30), axis=0, keepdims=True)
            smv = jnp.sum(jnp.where(mask, hb, 0.0), axis=0, keepdims=True)
            ctv = jnp.sum(mask.astype(jnp.float32))
            mx_ref[pl.ds(g, 1), :] = jnp.maximum(mx_ref[pl.ds(g, 1), :], mxv)
            sm_ref[pl.ds(g, 1), :] = sm_ref[pl.ds(g, 1), :] + smv
            ct_ref[pl.ds(g, 1), :] = ct_ref[pl.ds(g, 1), :] + ctv

    grid_spec = pltpu.PrefetchScalarGridSpec(
        num_scalar_prefetch=2,
        grid=(nblk, G),
        in_specs=[pl.BlockSpec((bm, D), lambda i, g, *_: (i, 0)),
                  pl.BlockSpec((bm, 1), lambda i, g, *_: (i, 0))],
        out_specs=[pl.BlockSpec((G, D), lambda i, g, *_: (0, 0)),
                   pl.BlockSpec((G, D), lambda i, g, *_: (0, 0)),
                   pl.BlockSpec((G, 128), lambda i, g, *_: (0, 0))])
    return pl.pallas_call(
        body, grid_spec=grid_spec,
        out_shape=[jax.ShapeDtypeStruct((G, D), jnp.float32),
                   jax.ShapeDtypeStruct((G, D), jnp.float32),
                   jax.ShapeDtypeStruct((G, 128), jnp.float32)],
    )(bmin, bmax, h, batch2d)


def final_mlp(mx, sm, cnt, ln_g, ln_b, w1, b1, w2, b2):
    D = mx.shape[-1]
    H2 = w2.shape[1]

    def body(mx_ref, sm_ref, cnt_ref, lng_ref, lnb_ref,
             w1_ref, b1_ref, w2_ref, b2_ref, o_ref):
        c = cnt_ref[:, 0:1]
        meanp = sm_ref[...] / jnp.maximum(c, 1.0)
        maxp = jnp.where(c > 0, mx_ref[...], 0.0)
        z = jnp.concatenate([maxp, meanp], axis=1)
        mu = jnp.mean(z, axis=-1, keepdims=True)
        var = jnp.mean((z - mu) ** 2, axis=-1, keepdims=True)
        z = (z - mu) / jnp.sqrt(var + 1e-5) * lng_ref[...] + lnb_ref[...]
        z = _leaky(jnp.dot(z, w1_ref[...],
                           preferred_element_type=jnp.float32) + b1_ref[...])
        z = jnp.dot(z, w2_ref[...],
                    preferred_element_type=jnp.float32) + b2_ref[...]
        o_ref[...] = jax.nn.sigmoid(z)

    return pl.pallas_call(
        body,
        out_shape=jax.ShapeDtypeStruct((G, H2), jnp.float32),
    )(mx, sm, cnt, ln_g.reshape(1, 2 * D), ln_b.reshape(1, 2 * D),
      w1, b1.reshape(1, w1.shape[1]), w2, b2.reshape(1, H2))


def _barrier():
    plsc.subcore_barrier()


def _axis_ids():
    return lax.axis_index("c"), lax.axis_index("s")


def _vperm(v, idx):
    return lax.gather(
        v, idx.reshape(16, 1),
        lax.GatherDimensionNumbers(offset_dims=(), collapsed_slice_dims=(0,),
                                   start_index_map=(0,)),
        (1,), mode=lax.GatherScatterMode.PROMISE_IN_BOUNDS)


def _lane_sum(v):
    iota = lax.broadcasted_iota(jnp.int32, (16,), 0)
    for sh in (8, 4, 2, 1):
        v = v + _vperm(v, iota ^ sh)
    return v


def _gather_rows(tab_h, idx_ref, out_ref, sem):
    pltpu.async_copy(tab_h.at[idx_ref], out_ref, sem).wait()


def _copy_start(src, dst, sem):
    return pltpu.async_copy(src, dst, sem)


def _gather_rows2(tab1, idx1, out1, sem1, tab2, idx2, out2, sem2):
    d1 = pltpu.async_copy(tab1.at[idx1], out1, sem1)
    d2 = pltpu.async_copy(tab2.at[idx2], out2, sem2)
    d1.wait()
    d2.wait()


def _scatter_add_rows(src_ref, base_ref, idx_ref):
    pltpu.sync_copy(src_ref, base_ref.at[idx_ref], add=True)


def _zero_vec_buf(buf, rows):
    def zi(i, _):
        buf[i] = jnp.zeros((16,), jnp.float32)
        return 0
    lax.fori_loop(0, rows, zi, 0, unroll=False)


@functools.lru_cache(maxsize=None)
def _make_gat_w(e_pad, H):
    epw = e_pad // NW
    B = 32
    nblk = epw // B
    assert nblk % 2 == 0
    zsl = N_PAD // NS

    @functools.partial(
        pl.kernel, mesh=_mesh(),
        out_type=[jax.ShapeDtypeStruct((e_pad, 16), jnp.float32),
                  jax.ShapeDtypeStruct((NC, N_PAD, 128), jnp.float32)],
        scratch_types=[
            [pltpu.VMEM((B,), jnp.int32)] * 2,
            [pltpu.VMEM((B,), jnp.int32)] * 2,
            [pltpu.VMEM((B, 128), jnp.float32)] * 2,
            [pltpu.VMEM((B, 128), jnp.float32)] * 2,
            pltpu.VMEM((B, 16), jnp.float32),
            pltpu.VMEM((B, 128), jnp.float32),
            pltpu.VMEM((8, 128), jnp.float32),
            pltpu.VMEM_SHARED((N_PAD, 128), jnp.float32),
            [pltpu.SemaphoreType.DMA] * 2,
            [pltpu.SemaphoreType.DMA] * 2,
            [pltpu.SemaphoreType.DMA] * 2,
            [pltpu.SemaphoreType.DMA] * 2,
        ])
    def k(src_h, dst_h, sc_h, w_h, den_h,
          srcv, dstv, ur, vr, wb, wwide, zb, dacc, semA, semB, sem1, sem2):
        cid, sid = _axis_ids()
        wid = sid * NC + cid

        def zrow(i, _):
            for j in range(8):
                zb[i, pl.ds(j * 16, 16)] = jnp.zeros((16,), jnp.float32)
            return 0
        lax.fori_loop(0, 8, zrow, 0, unroll=False)

        def zwide(i, _):
            for j in range(8):
                wwide[i, pl.ds(j * 16, 16)] = jnp.zeros((16,), jnp.float32)
            return 0
        lax.fori_loop(0, B, zwide, 0, unroll=False)

        def zcp(zi, _):
            pltpu.sync_copy(zb, dacc.at[pl.ds(sid * zsl + zi * 8, 8)])
            return 0
        lax.fori_loop(0, zsl // 8, zcp, 0, unroll=False)
        _barrier()
        base0 = wid * epw

        def stage(p, base):
            d1 = _copy_start(src_h.at[pl.ds(base, B)], srcv[p], semA[p])
            d2 = _copy_start(dst_h.at[pl.ds(base, B)], dstv[p], semB[p])
            d1.wait()
            d2.wait()
            dg1 = _copy_start(sc_h.at[srcv[p]], ur[p], sem1[p])
            dg2 = _copy_start(sc_h.at[dstv[p]], vr[p], sem2[p])
            return dg1, dg2

        def drain(p, base, dg1, dg2):
            dg1.wait()
            dg2.wait()

            def per_edge(e, _):
                a = ur[p][e, pl.ds(0, 16)] + vr[p][e, pl.ds(16, 16)]
                w = jnp.exp(jnp.where(a >= 0, a, 0.2 * a))
                wb[e] = w
                wwide[e, pl.ds(0, 16)] = w
                return 0
            lax.fori_loop(0, B, per_edge, 0, unroll=False)
            pltpu.sync_copy(wb, w_h.at[pl.ds(base, B)])
            _scatter_add_rows(wwide, dacc, dstv[p])

        def blk(pi, _):
            base = base0 + pi * 2 * B
            da = stage(0, base)
            db = stage(1, base + B)
            drain(0, base, *da)
            drain(1, base + B, *db)
            return 0
        lax.fori_loop(0, nblk // 2, blk, 0, unroll=False)
        _barrier()
        pltpu.sync_copy(dacc.at[pl.ds(sid * zsl, zsl)],
                        den_h.at[cid, pl.ds(sid * zsl, zsl)])

    return k


@functools.lru_cache(maxsize=None)
def _make_trans_w(e_pad, H, C, B):
    D = H * C
    epw = e_pad // NW
    nblk = epw // B
    assert nblk % 2 == 0
    zsl = N_PAD // NS
    scale = 1.0 / math.sqrt(float(C))

    @functools.partial(
        pl.kernel, mesh=_mesh(),
        out_type=[jax.ShapeDtypeStruct((e_pad, 16), jnp.float32),
                  jax.ShapeDtypeStruct((NC, N_PAD, 128), jnp.float32)],
        scratch_types=[
            [pltpu.VMEM((B,), jnp.int32)] * 2,
            [pltpu.VMEM((B,), jnp.int32)] * 2,
            [pltpu.VMEM((B, D), jnp.float32)] * 2,
            [pltpu.VMEM((B, D), jnp.float32)] * 2,
            pltpu.VMEM((B, 16), jnp.float32),
            pltpu.VMEM((B, 128), jnp.float32),
            pltpu.VMEM((8, 128), jnp.float32),
            pltpu.VMEM_SHARED((N_PAD, 128), jnp.float32),
            [pltpu.SemaphoreType.DMA] * 2,
            [pltpu.SemaphoreType.DMA] * 2,
            [pltpu.SemaphoreType.DMA] * 2,
            [pltpu.SemaphoreType.DMA] * 2,
        ])
    def k(src_h, dst_h, q_h, k_h, w_h, den_h,
          srcv, dstv, qr, kr, wb, wwide, zb, dacc, semA, semB, sem1, sem2):
        cid, sid = _axis_ids()
        wid = sid * NC + cid
        iota = lax.broadcasted_iota(jnp.int32, (16,), 0)

        def zrow(i, _):
            for j in range(8):
                zb[i, pl.ds(j * 16, 16)] = jnp.zeros((16,), jnp.float32)
            return 0
        lax.fori_loop(0, 8, zrow, 0, unroll=False)

        def zwide(i, _):
            for j in range(8):
                wwide[i, pl.ds(j * 16, 16)] = jnp.zeros((16,), jnp.float32)
            return 0
        lax.fori_loop(0, B, zwide, 0, unroll=False)

        def zcp(zi, _):
            pltpu.sync_copy(zb, dacc.at[pl.ds(sid * zsl + zi * 8, 8)])
            return 0
        lax.fori_loop(0, zsl // 8, zcp, 0, unroll=False)
        _barrier()
        base0 = wid * epw

        def stage(p, base):
            d1 = _copy_start(src_h.at[pl.ds(base, B)], srcv[p], semA[p])
            d2 = _copy_start(dst_h.at[pl.ds(base, B)], dstv[p], semB[p])
            d1.wait()
            d2.wait()
            dg1 = _copy_start(q_h.at[dstv[p]], qr[p], sem1[p])
            dg2 = _copy_start(k_h.at[srcv[p]], kr[p], sem2[p])
            return dg1, dg2

        def drain(p, base, dg1, dg2):
            dg1.wait()
            dg2.wait()

            def per_edge(e, _):
                w = jnp.zeros((16,), jnp.float32)
                for hd in range(H):
                    acc = jnp.zeros((16,), jnp.float32)
                    for j in range(C // 16):
                        off = hd * C + j * 16
                        acc = acc + (qr[p][e, pl.ds(off, 16)] *
                                     kr[p][e, pl.ds(off, 16)])
                    t = _lane_sum(acc) * scale
                    w = jnp.where(iota == hd, t, w)
                w = jnp.exp(jnp.where(iota < H, w, jnp.zeros((16,), jnp.float32)))
                w = jnp.where(iota < H, w, jnp.zeros((16,), jnp.float32))
                wb[e] = w
                wwide[e, pl.ds(0, 16)] = w
                return 0
            lax.fori_loop(0, B, per_edge, 0, unroll=False)
            pltpu.sync_copy(wb, w_h.at[pl.ds(base, B)])
            _scatter_add_rows(wwide, dacc, dstv[p])

        def blk(pi, _):
            base = base0 + pi * 2 * B
            da = stage(0, base)
            db = stage(1, base + B)
            drain(0, base, *da)
            drain(1, base + B, *db)
            return 0
        lax.fori_loop(0, nblk // 2, blk, 0, unroll=False)
        _barrier()
        pltpu.sync_copy(dacc.at[pl.ds(sid * zsl, zsl)],
                        den_h.at[cid, pl.ds(sid * zsl, zsl)])

    return k


@functools.lru_cache(maxsize=None)
def _make_agg(e_pad, K, C):
    KPC = K // NC
    epw = e_pad // NS
    B = 64
    nblk = epw // B
    assert nblk % 2 == 0
    zsl = N_PAD // NS
    logc = int(math.log2(C))

    @functools.partial(
        pl.kernel, mesh=_mesh(),
        out_type=jax.ShapeDtypeStruct((K, N_PAD, 128), jnp.float32),
        scratch_types=[
            [pltpu.VMEM((B,), jnp.int32)] * 2,
            [pltpu.VMEM((B,), jnp.int32)] * 2,
            [pltpu.VMEM((B,), jnp.int32)] * 2,
            [pltpu.VMEM((B, 128), jnp.float32)] * 2,
            [pltpu.VMEM((B, 16), jnp.float32)] * 2,
            pltpu.VMEM((16, 128), jnp.float32),
            pltpu.VMEM_SHARED((N_PAD, 128), jnp.float32),
            [pltpu.SemaphoreType.DMA] * 2,
            [pltpu.SemaphoreType.DMA] * 2,
            [pltpu.SemaphoreType.DMA] * 2,
            [pltpu.SemaphoreType.DMA] * 2,
        ])
    def k(src_h, dst_h, w_h, v_h, out_h,
          srcv, dstv, idxv, rows, wb, zb, acc, semg, semA, semB, semC):
        cid, sid = _axis_ids()

        def zrow(i, _):
            for j in range(8):
                zb[i, pl.ds(j * 16, 16)] = jnp.zeros((16,), jnp.float32)
            return 0
        lax.fori_loop(0, 16, zrow, 0, unroll=False)

        for kk in range(KPC):
            kchunk = cid * KPC + kk
            kbase = kchunk * N_PAD

            def zcp(zi, _):
                pltpu.sync_copy(zb, acc.at[pl.ds(sid * zsl + zi * 16, 16)])
                return 0
            lax.fori_loop(0, zsl // 16, zcp, 0, unroll=False)
            _barrier()

            def stage(p, base):
                d1 = _copy_start(src_h.at[pl.ds(base, B)], srcv[p], semA[p])
                d2 = _copy_start(dst_h.at[pl.ds(base, B)], dstv[p], semB[p])
                d3 = _copy_start(w_h.at[pl.ds(base, B)], wb[p], semC[p])
                d1.wait()

                def mkidx(i, _):
                    idxv[p][pl.ds(i * 16, 16)] = (
                        srcv[p][pl.ds(i * 16, 16)] + kbase)
                    return 0
                lax.fori_loop(0, B // 16, mkidx, 0, unroll=True)
                dg = _copy_start(v_h.at[idxv[p]], rows[p], semg[p])
                return d2, d3, dg

            def drain(p, d2, d3, dg):
                d3.wait()
                dg.wait()

                def per_edge(e, _):
                    wv = wb[p][e]
                    for j in range(8):
                        hdj = (kchunk * 128 + j * 16) >> logc
                        m = _vperm(wv, jnp.full((16,), hdj, jnp.int32))
                        rows[p][e, pl.ds(j * 16, 16)] = (
                            rows[p][e, pl.ds(j * 16, 16)] * m)
                    return 0
                lax.fori_loop(0, B, per_edge, 0, unroll=False)
                d2.wait()
                _scatter_add_rows(rows[p], acc, dstv[p])

            def blk(pi, _):
                base = sid * epw + pi * 2 * B
                da = stage(0, base)
                db = stage(1, base + B)
                drain(0, *da)
                drain(1, *db)
                return 0
            lax.fori_loop(0, nblk // 2, blk, 0, unroll=False)
            _barrier()
            pltpu.sync_copy(acc.at[pl.ds(sid * zsl, zsl)],
                            out_h.at[kchunk, pl.ds(sid * zsl, zsl)])
            _barrier()

    return k


def _score_weights(W, a_s, a_d, H, C):
    Din = W.shape[0]
    ws = (W.reshape(Din, H, C) * a_s[None]).sum(-1)
    wd = (W.reshape(Din, H, C) * a_d[None]).sum(-1)
    Wsc = jnp.zeros((Din, 128), jnp.float32)
    Wsc = Wsc.at[:, 0:H].set(ws).at[:, 16:16 + H].set(wd)
    return Wsc


def _gat_layer(h, src, dst, W, a_s, a_d, b, bn_g, bn_b, bn_m, bn_v,
               H, C, concat):
    Kc = (H * C) // 128
    hp = mm_chunk(h, W, jnp.zeros((H * C,), jnp.float32))
    sc = mm_flat(h, _score_weights(W, a_s, a_d, H, C),
                 jnp.zeros((128,), jnp.float32), bn=128)
    w, den = _make_gat_w(E_GAT, H)(src, dst, sc)
    num = _make_agg(E_GAT, Kc, C)(src, dst, w, hp.reshape(Kc * N_PAD, 128))
    return gat_post(num, den, b, bn_g, bn_b, bn_m, bn_v, H, C, concat)


def _trans_layer(h, src, dst, Wq, bq, Wk, bk, Wv, bv, Ws, bs,
                 H, C, concat, identity):
    Kc = (H * C) // 128
    q = mm_flat(h, Wq, bq)
    kt = mm_flat(h, Wk, bk)
    v = mm_chunk(h, Wv, bv)
    skip = mm_flat(h, Ws, bs)
    B = 16 if H * C <= 512 else 8
    w, den = _make_trans_w(E_TRN, H, C, B)(src, dst, q, kt)
    num = _make_agg(E_TRN, Kc, C)(src, dst, w, v.reshape(Kc * N_PAD, 128))
    return trans_post(num, den, skip, identity, H, C, concat)


def kernel(x, edge_index, batch, params):
    p = params
    src = edge_index[0].astype(jnp.int32)
    dst = edge_index[1].astype(jnp.int32)
    loop = jnp.arange(N, dtype=jnp.int32)
    fill_g = jnp.full((E_GAT - E - N,), _DUMMY, jnp.int32)
    src_g = jnp.concatenate([src, loop, fill_g])
    dst_g = jnp.concatenate([dst, loop, fill_g])
    fill_t = jnp.full((E_TRN - E,), _DUMMY, jnp.int32)
    src_t = jnp.concatenate([src, fill_t])
    dst_t = jnp.concatenate([dst, fill_t])

    xp = jnp.zeros((N_PAD, x.shape[1]), jnp.float32).at[:N].set(x)
    batch_p = jnp.concatenate(
        [batch.astype(jnp.int32), jnp.full((N_PAD - N,), G, jnp.int32)])

    h1 = _gat_layer(xp, src_g, dst_g, p['W1'], p['as1'], p['ad1'], p['b1'],
                    p['bn1_g'], p['bn1_b'], p['bn1_m'], p['bn1_v'],
                    8, 32, True)
    h2 = _gat_layer(h1, src_g, dst_g, p['W2'], p['as2'], p['ad2'], p['b2'],
                    p['bn2_g'], p['bn2_b'], p['bn2_m'], p['bn2_v'],
                    8, 64, True)
    h3 = _gat_layer(h2, src_g, dst_g, p['W3'], p['as3'], p['ad3'], p['b3'],
                    p['bn3_g'], p['bn3_b'], p['bn3_m'], p['bn3_v'],
                    4, 256, False)
    h4 = _trans_layer(h3, src_t, dst_t,
                      p['t1_Wq'], p['t1_bq'], p['t1_Wk'], p['t1_bk'],
                      p['t1_Wv'], p['t1_bv'], p['t1_Ws'], p['t1_bs'],
                      8, 64, True, None)
    h5 = _trans_layer(h4, src_t, dst_t,
                      p['t2_Wq'], p['t2_bq'], p['t2_Wk'], p['t2_bk'],
                      p['t2_Wv'], p['t2_bv'], p['t2_Ws'], p['t2_bs'],
                      4, 256, False, h3)

    bmin = batch_p[0::256]
    bmax = batch_p[255::256]
    mx, sm, cnt = pool_tc(h5, batch_p.reshape(N_PAD, 1), bmin, bmax)
    return final_mlp(mx, sm, cnt,
                     p['ln_g'], p['ln_b'], p['fc1_W'], p['fc1_b'],
                     p['fc2_W'], p['fc2_b'])
